# Initial kernel scaffold; baseline (speedup 1.0000x reference)
import jax
import jax.numpy as jnp
from jax.experimental import pallas as pl


def kernel(features, edge_index, edge_types, c1_W, c1_b, c1_Wih, c1_Whh, c1_bih, c1_bhh, c2_W, c2_b, c2_Wih, c2_Whh, c2_bih, c2_bhh, W3, b3, W4, b4):
    raise NotImplementedError("write your pallas kernel here")


# speedup vs baseline: 9.0526x; 9.0526x over previous
import functools

import jax
import jax.numpy as jnp
from jax import lax
from jax.experimental import pallas as pl
from jax.experimental.pallas import tpu as pltpu, tpu_sc as plsc

N, E, D, H, K = 10000, 320000, 128, 256, 4

NC, NS, L = 2, 16, 16
NW = NC * NS

CHUNK = 128
EPW = 10112
NCH = EPW // CHUNK
EPAD = EPW * NW
NACC = 10112
RPT = NACC // NS
BP = 400

_SC_PARAMS = pltpu.CompilerParams(needs_layout_passes=False)


def _mesh():
    return plsc.VectorSubcoreMesh(
        core_axis_name="c", subcore_axis_name="s",
        num_cores=NC, num_subcores=NS)


def _seg_sum_body(proj_hbm, gidx_hbm, dst_hbm, zero_hbm, out_hbm,
                  gi_v, ds_v, rows_v, acc_sh, sem):
    c = lax.axis_index("c")
    s = lax.axis_index("s")
    wid = s * NC + c
    pltpu.sync_copy(zero_hbm.at[pl.ds(s * RPT, RPT)],
                    acc_sh.at[pl.ds(s * RPT, RPT)])
    plsc.subcore_barrier()

    def body(i, carry):
        base = wid * EPW + i * CHUNK
        pltpu.sync_copy(gidx_hbm.at[pl.ds(base, CHUNK)], gi_v)
        pltpu.sync_copy(dst_hbm.at[pl.ds(base, CHUNK)], ds_v)
        pltpu.async_copy(proj_hbm.at[gi_v], rows_v, sem).wait()
        pltpu.sync_copy(rows_v, acc_sh.at[ds_v], add=True)
        return carry

    lax.fori_loop(0, NCH, body, 0)
    plsc.subcore_barrier()
    pltpu.sync_copy(acc_sh.at[pl.ds(s * RPT, RPT)],
                    out_hbm.at[c].at[pl.ds(s * RPT, RPT)])


@functools.cache
def _seg_sum_kernel():
    return pl.kernel(
        _seg_sum_body,
        out_type=jax.ShapeDtypeStruct((NC, NACC, D), jnp.float32),
        mesh=_mesh(),
        scratch_types=[
            pltpu.VMEM((CHUNK,), jnp.int32),
            pltpu.VMEM((CHUNK,), jnp.int32),
            pltpu.VMEM((CHUNK, D), jnp.float32),
            pltpu.VMEM_SHARED((NACC, D), jnp.float32),
            pltpu.SemaphoreType.DMA,
        ],
        compiler_params=_SC_PARAMS,
    )


def _seg_sum(proj, gidx_p, dst_p, zeros_acc):
    return _seg_sum_kernel()(proj, gidx_p, dst_p, zeros_acc)


def _edge_score_body(x_hbm, src_hbm, dst_hbm, out_hbm,
                     si_v, di_v, rs_v, rd_v, o_v, sem):
    c = lax.axis_index("c")
    s = lax.axis_index("s")
    wid = s * NC + c
    n_groups = CHUNK // L

    def body(i, carry):
        base = wid * EPW + i * CHUNK
        pltpu.sync_copy(src_hbm.at[pl.ds(base, CHUNK)], si_v)
        pltpu.sync_copy(dst_hbm.at[pl.ds(base, CHUNK)], di_v)
        pltpu.async_copy(x_hbm.at[si_v], rs_v, sem).wait()
        pltpu.async_copy(x_hbm.at[di_v], rd_v, sem).wait()

        def fbody(f, accs):
            fs = jnp.full((L,), 0, jnp.int32) + f
            new = []
            for g in range(n_groups):
                rows = g * L + lax.iota(jnp.int32, L)
                vs = plsc.load_gather(rs_v, [rows, fs])
                vd = plsc.load_gather(rd_v, [rows, fs])
                new.append(accs[g] + vs * vd)
            return tuple(new)

        accs = lax.fori_loop(
            0, D, fbody,
            tuple(jnp.zeros((L,), jnp.float32) for _ in range(n_groups)))
        for g in range(n_groups):
            t = accs[g]
            e1 = 1.0 / (1.0 + jnp.exp(-t))
            e2 = 1.0 / (1.0 + jnp.exp(-e1))
            o_v[pl.ds(g * L, L)] = e2
        pltpu.sync_copy(o_v, out_hbm.at[pl.ds(base, CHUNK)])
        return carry

    lax.fori_loop(0, NCH, body, 0)


@functools.cache
def _edge_score_kernel():
    return pl.kernel(
        _edge_score_body,
        out_type=jax.ShapeDtypeStruct((EPAD,), jnp.float32),
        mesh=_mesh(),
        scratch_types=[
            pltpu.VMEM((CHUNK,), jnp.int32),
            pltpu.VMEM((CHUNK,), jnp.int32),
            pltpu.VMEM((CHUNK, D), jnp.float32),
            pltpu.VMEM((CHUNK, D), jnp.float32),
            pltpu.VMEM((CHUNK,), jnp.float32),
            pltpu.SemaphoreType.DMA,
        ],
        compiler_params=_SC_PARAMS,
    )


def _edge_score(x, src_sp, dst_sp):
    return _edge_score_kernel()(x, src_sp, dst_sp)


def _proj_body(h_ref, wt_ref, b_ref, out_ref):
    h = h_ref[...]
    for k in range(K):
        out_ref[k] = (jnp.dot(h, wt_ref[k], preferred_element_type=jnp.float32)
                      + b_ref[k])


def _tc_proj(h, wt, b3d):
    return pl.pallas_call(
        _proj_body,
        grid=(N // BP,),
        in_specs=[
            pl.BlockSpec((BP, D), lambda i: (i, 0)),
            pl.BlockSpec((K, D, D), lambda i: (0, 0, 0)),
            pl.BlockSpec((K, 1, D), lambda i: (0, 0, 0)),
        ],
        out_specs=pl.BlockSpec((K, BP, D), lambda i: (0, i, 0)),
        out_shape=jax.ShapeDtypeStruct((K, N, D), jnp.float32),
    )(h, wt, b3d)


def _gru_body(relu, p_ref, h_ref, wih_ref, whh_ref, bih_ref, bhh_ref, out_ref):
    a = p_ref[0] + p_ref[1]
    h = h_ref[...]
    gi = jnp.dot(a, wih_ref[...], preferred_element_type=jnp.float32) + bih_ref[...]
    gh = jnp.dot(h, whh_ref[...], preferred_element_type=jnp.float32) + bhh_ref[...]
    r = jax.nn.sigmoid(gi[:, :D] + gh[:, :D])
    z = jax.nn.sigmoid(gi[:, D:2 * D] + gh[:, D:2 * D])
    n = jnp.tanh(gi[:, 2 * D:] + r * gh[:, 2 * D:])
    hn = (1.0 - z) * n + z * h
    if relu:
        hn = jnp.maximum(hn, 0.0)
    out_ref[...] = hn


def _tc_gru(partials, h, wih_t, whh_t, bih2, bhh2, relu):
    return pl.pallas_call(
        functools.partial(_gru_body, relu),
        grid=(N // BP,),
        in_specs=[
            pl.BlockSpec((NC, BP, D), lambda i: (0, i, 0)),
            pl.BlockSpec((BP, D), lambda i: (i, 0)),
            pl.BlockSpec((D, 3 * D), lambda i: (0, 0)),
            pl.BlockSpec((D, 3 * D), lambda i: (0, 0)),
            pl.BlockSpec((1, 3 * D), lambda i: (0, 0)),
            pl.BlockSpec((1, 3 * D), lambda i: (0, 0)),
        ],
        out_specs=pl.BlockSpec((BP, D), lambda i: (i, 0)),
        out_shape=jax.ShapeDtypeStruct((N, D), jnp.float32),
    )(partials, h, wih_t, whh_t, bih2, bhh2)


def _mlp_body(x_ref, w3_ref, b3_ref, w4_ref, b4_ref, out_ref):
    x = x_ref[...]
    y = jnp.maximum(
        jnp.dot(x, w3_ref[...], preferred_element_type=jnp.float32) + b3_ref[...],
        0.0)
    z = jnp.maximum(
        jnp.dot(y, w4_ref[...], preferred_element_type=jnp.float32) + b4_ref[...],
        0.0)
    out_ref[...] = z


def _tc_mlp(x, w3t, b32, w4t, b42):
    return pl.pallas_call(
        _mlp_body,
        grid=(N // BP,),
        in_specs=[
            pl.BlockSpec((BP, D), lambda i: (i, 0)),
            pl.BlockSpec((D, H), lambda i: (0, 0)),
            pl.BlockSpec((1, H), lambda i: (0, 0)),
            pl.BlockSpec((H, D), lambda i: (0, 0)),
            pl.BlockSpec((1, D), lambda i: (0, 0)),
        ],
        out_specs=pl.BlockSpec((BP, D), lambda i: (i, 0)),
        out_shape=jax.ShapeDtypeStruct((N, D), jnp.float32),
    )(x, w3t, b32, w4t, b42)


def kernel(features, edge_index, edge_types, c1_W, c1_b, c1_Wih, c1_Whh,
           c1_bih, c1_bhh, c2_W, c2_b, c2_Wih, c2_Whh, c2_bih, c2_bhh,
           W3, b3, W4, b4):
    src = edge_index[0]
    dst = edge_index[1]
    gidx = edge_types * N + src

    npad = EPAD - E
    pad_ids = jnp.arange(npad, dtype=jnp.int32)
    gidx_p = jnp.concatenate([gidx, pad_ids % 997])
    dst_p = jnp.concatenate([dst, N + (pad_ids % (NACC - N))])
    src_sp = jnp.concatenate([src, pad_ids % 997])
    dst_sp = jnp.concatenate([dst, (pad_ids + 499) % 997])

    zeros_acc = jnp.zeros((NACC, D), jnp.float32)

    def ggc(h, Wc, bc, Wih, Whh, bih, bhh):
        wt = jnp.transpose(Wc, (0, 2, 1))
        b3d = bc[:, None, :]
        wih_t = Wih.T
        whh_t = Whh.T
        bih2 = bih[None, :]
        bhh2 = bhh[None, :]
        for step in range(2):
            proj = _tc_proj(h, wt, b3d).reshape(K * N, D)
            partials = _seg_sum(proj, gidx_p, dst_p, zeros_acc)
            h = _tc_gru(partials, h, wih_t, whh_t, bih2, bhh2,
                        relu=(step == 1))
        return h

    h = ggc(features, c1_W, c1_b, c1_Wih, c1_Whh, c1_bih, c1_bhh)
    h = ggc(h, c2_W, c2_b, c2_Wih, c2_Whh, c2_bih, c2_bhh)
    x = _tc_mlp(h, W3.T, b3[None, :], W4.T, b4[None, :])
    scores = _edge_score(x, src_sp, dst_sp)
    return scores[:E]


# speedup vs baseline: 27.4079x; 3.0276x over previous
import functools

import jax
import jax.numpy as jnp
from jax import lax
from jax.experimental import pallas as pl
from jax.experimental.pallas import tpu as pltpu, tpu_sc as plsc

N, E, D, H, K = 10000, 320000, 128, 256, 4

NC, NS, L = 2, 16, 16
NW = NC * NS

CHUNK = 128
NCH = 80
EPW = NCH * CHUNK
EPAD = EPW * NW
NACC = 10112
RPT = NACC // NS
BP = 400
PROWS = EPAD // 8

_SC_PARAMS = pltpu.CompilerParams(needs_layout_passes=False)


def _mesh():
    return plsc.VectorSubcoreMesh(
        core_axis_name="c", subcore_axis_name="s",
        num_cores=NC, num_subcores=NS)


def _seg_sum_body(proj_hbm, gidx_hbm, dst_hbm, zero_hbm, out_hbm,
                  gi0, gi1, di0, di1, rows0, rows1, acc_sh,
                  sg0, sg1, sgi0, sgi1, sdi0, sdi1):
    c = lax.axis_index("c")
    s = lax.axis_index("s")
    wid = s * NC + c
    base0 = wid * EPW
    gis = (gi0, gi1)
    dis = (di0, di1)
    rows = (rows0, rows1)
    sg = (sg0, sg1)
    sgi = (sgi0, sgi1)
    sdi = (sdi0, sdi1)
    pltpu.sync_copy(zero_hbm.at[pl.ds(s * RPT, RPT)],
                    acc_sh.at[pl.ds(s * RPT, RPT)])
    plsc.subcore_barrier()

    pltpu.sync_copy(gidx_hbm.at[pl.ds(base0, CHUNK)], gi0)
    pltpu.sync_copy(dst_hbm.at[pl.ds(base0, CHUNK)], di0)
    pltpu.async_copy(proj_hbm.at[gi0], rows0, sg0)
    pltpu.async_copy(gidx_hbm.at[pl.ds(base0 + CHUNK, CHUNK)], gi1, sgi1)
    pltpu.async_copy(dst_hbm.at[pl.ds(base0 + CHUNK, CHUNK)], di1, sdi1)

    def body(i, carry):
        for b in range(2):
            cc = 2 * i + b

            @pl.when(cc + 1 < NCH)
            def _():
                pltpu.make_async_copy(
                    gidx_hbm.at[pl.ds(base0 + (cc + 1) * CHUNK, CHUNK)],
                    gis[1 - b], sgi[1 - b]).wait()
                pltpu.make_async_copy(
                    dst_hbm.at[pl.ds(base0 + (cc + 1) * CHUNK, CHUNK)],
                    dis[1 - b], sdi[1 - b]).wait()
                pltpu.async_copy(proj_hbm.at[gis[1 - b]], rows[1 - b],
                                 sg[1 - b])

            pltpu.make_async_copy(proj_hbm.at[gis[b]], rows[b], sg[b]).wait()

            @pl.when(cc + 2 < NCH)
            def _():
                pltpu.async_copy(
                    gidx_hbm.at[pl.ds(base0 + (cc + 2) * CHUNK, CHUNK)],
                    gis[b], sgi[b])

            pltpu.sync_copy(rows[b], acc_sh.at[dis[b]], add=True)

            @pl.when(cc + 2 < NCH)
            def _():
                pltpu.async_copy(
                    dst_hbm.at[pl.ds(base0 + (cc + 2) * CHUNK, CHUNK)],
                    dis[b], sdi[b])
        return carry

    lax.fori_loop(0, NCH // 2, body, 0)
    plsc.subcore_barrier()
    pltpu.sync_copy(acc_sh.at[pl.ds(s * RPT, RPT)],
                    out_hbm.at[c].at[pl.ds(s * RPT, RPT)])


@functools.cache
def _seg_sum_kernel():
    return pl.kernel(
        _seg_sum_body,
        out_type=jax.ShapeDtypeStruct((NC, NACC, D), jnp.float32),
        mesh=_mesh(),
        scratch_types=[
            pltpu.VMEM((CHUNK,), jnp.int32),
            pltpu.VMEM((CHUNK,), jnp.int32),
            pltpu.VMEM((CHUNK,), jnp.int32),
            pltpu.VMEM((CHUNK,), jnp.int32),
            pltpu.VMEM((CHUNK, D), jnp.float32),
            pltpu.VMEM((CHUNK, D), jnp.float32),
            pltpu.VMEM_SHARED((NACC, D), jnp.float32),
            pltpu.SemaphoreType.DMA,
            pltpu.SemaphoreType.DMA,
            pltpu.SemaphoreType.DMA,
            pltpu.SemaphoreType.DMA,
            pltpu.SemaphoreType.DMA,
            pltpu.SemaphoreType.DMA,
        ],
        compiler_params=_SC_PARAMS,
    )


def _seg_sum(proj, gidx_p, dst_p, zeros_acc):
    return _seg_sum_kernel()(proj, gidx_p, dst_p, zeros_acc)


def _edge_score_body(x_hbm, src_hbm, dst_hbm, out_hbm,
                     si_v, di_v, rs0, rs1, rd0, rd1, p_v,
                     sr0, sr1, sd0, sd1):
    c = lax.axis_index("c")
    s = lax.axis_index("s")
    wid = s * NC + c
    pltpu.sync_copy(src_hbm.at[wid], si_v)
    pltpu.sync_copy(dst_hbm.at[wid], di_v)
    rsb = (rs0, rs1)
    rdb = (rd0, rd1)
    srs = (sr0, sr1)
    sds = (sd0, sd1)
    pltpu.async_copy(x_hbm.at[si_v.at[0]], rs0, sr0)
    pltpu.async_copy(x_hbm.at[di_v.at[0]], rd0, sd0)

    def body(i, carry):
        for b in range(2):
            cc = 2 * i + b
            pltpu.make_async_copy(x_hbm.at[si_v.at[cc]], rsb[b], srs[b]).wait()
            pltpu.make_async_copy(x_hbm.at[di_v.at[cc]], rdb[b], sds[b]).wait()

            @pl.when(cc + 1 < NCH)
            def _():
                pltpu.async_copy(x_hbm.at[si_v.at[cc + 1]],
                                 rsb[1 - b], srs[1 - b])
                pltpu.async_copy(x_hbm.at[di_v.at[cc + 1]],
                                 rdb[1 - b], sds[1 - b])

            rs, rd = rsb[b], rdb[b]

            def gbody(g, carry2):
                for j in range(8):
                    e = g * 8 + j
                    acc = rs[e, pl.ds(0, L)] * rd[e, pl.ds(0, L)]
                    for fc in range(1, 8):
                        acc = acc + (rs[e, pl.ds(fc * L, L)]
                                     * rd[e, pl.ds(fc * L, L)])
                    p_v[g, pl.ds(j * L, L)] = acc
                return carry2

            lax.fori_loop(0, CHUNK // 8, gbody, 0)
            base_row = wid * (EPW // 8) + cc * (CHUNK // 8)
            pltpu.sync_copy(p_v, out_hbm.at[pl.ds(base_row, CHUNK // 8)])
        return carry

    lax.fori_loop(0, NCH // 2, body, 0)


@functools.cache
def _edge_score_kernel():
    return pl.kernel(
        _edge_score_body,
        out_type=jax.ShapeDtypeStruct((PROWS, D), jnp.float32),
        mesh=_mesh(),
        scratch_types=[
            pltpu.VMEM((NCH, CHUNK), jnp.int32),
            pltpu.VMEM((NCH, CHUNK), jnp.int32),
            pltpu.VMEM((CHUNK, D), jnp.float32),
            pltpu.VMEM((CHUNK, D), jnp.float32),
            pltpu.VMEM((CHUNK, D), jnp.float32),
            pltpu.VMEM((CHUNK, D), jnp.float32),
            pltpu.VMEM((CHUNK // 8, D), jnp.float32),
            pltpu.SemaphoreType.DMA,
            pltpu.SemaphoreType.DMA,
            pltpu.SemaphoreType.DMA,
            pltpu.SemaphoreType.DMA,
        ],
        compiler_params=_SC_PARAMS,
    )


def _edge_score(x, src_sp, dst_sp):
    return _edge_score_kernel()(x, src_sp, dst_sp)


def _proj_body(h_ref, wt_ref, b_ref, out_ref):
    h = h_ref[...]
    for k in range(K):
        out_ref[k] = (jnp.dot(h, wt_ref[k], preferred_element_type=jnp.float32)
                      + b_ref[k])


def _tc_proj(h, wt, b3d):
    return pl.pallas_call(
        _proj_body,
        grid=(N // BP,),
        in_specs=[
            pl.BlockSpec((BP, D), lambda i: (i, 0)),
            pl.BlockSpec((K, D, D), lambda i: (0, 0, 0)),
            pl.BlockSpec((K, 1, D), lambda i: (0, 0, 0)),
        ],
        out_specs=pl.BlockSpec((K, BP, D), lambda i: (0, i, 0)),
        out_shape=jax.ShapeDtypeStruct((K, N, D), jnp.float32),
    )(h, wt, b3d)


def _gru_body(relu, p_ref, h_ref, wih_ref, whh_ref, bih_ref, bhh_ref, out_ref):
    a = p_ref[0] + p_ref[1]
    h = h_ref[...]
    gi = jnp.dot(a, wih_ref[...], preferred_element_type=jnp.float32) + bih_ref[...]
    gh = jnp.dot(h, whh_ref[...], preferred_element_type=jnp.float32) + bhh_ref[...]
    r = jax.nn.sigmoid(gi[:, :D] + gh[:, :D])
    z = jax.nn.sigmoid(gi[:, D:2 * D] + gh[:, D:2 * D])
    n = jnp.tanh(gi[:, 2 * D:] + r * gh[:, 2 * D:])
    hn = (1.0 - z) * n + z * h
    if relu:
        hn = jnp.maximum(hn, 0.0)
    out_ref[...] = hn


def _tc_gru(partials, h, wih_t, whh_t, bih2, bhh2, relu):
    return pl.pallas_call(
        functools.partial(_gru_body, relu),
        grid=(N // BP,),
        in_specs=[
            pl.BlockSpec((NC, BP, D), lambda i: (0, i, 0)),
            pl.BlockSpec((BP, D), lambda i: (i, 0)),
            pl.BlockSpec((D, 3 * D), lambda i: (0, 0)),
            pl.BlockSpec((D, 3 * D), lambda i: (0, 0)),
            pl.BlockSpec((1, 3 * D), lambda i: (0, 0)),
            pl.BlockSpec((1, 3 * D), lambda i: (0, 0)),
        ],
        out_specs=pl.BlockSpec((BP, D), lambda i: (i, 0)),
        out_shape=jax.ShapeDtypeStruct((N, D), jnp.float32),
    )(partials, h, wih_t, whh_t, bih2, bhh2)


def _mlp_body(x_ref, w3_ref, b3_ref, w4_ref, b4_ref, out_ref):
    x = x_ref[...]
    y = jnp.maximum(
        jnp.dot(x, w3_ref[...], preferred_element_type=jnp.float32) + b3_ref[...],
        0.0)
    z = jnp.maximum(
        jnp.dot(y, w4_ref[...], preferred_element_type=jnp.float32) + b4_ref[...],
        0.0)
    out_ref[...] = z


def _tc_mlp(x, w3t, b32, w4t, b42):
    return pl.pallas_call(
        _mlp_body,
        grid=(N // BP,),
        in_specs=[
            pl.BlockSpec((BP, D), lambda i: (i, 0)),
            pl.BlockSpec((D, H), lambda i: (0, 0)),
            pl.BlockSpec((1, H), lambda i: (0, 0)),
            pl.BlockSpec((H, D), lambda i: (0, 0)),
            pl.BlockSpec((1, D), lambda i: (0, 0)),
        ],
        out_specs=pl.BlockSpec((BP, D), lambda i: (i, 0)),
        out_shape=jax.ShapeDtypeStruct((N, D), jnp.float32),
    )(x, w3t, b32, w4t, b42)


def _fin_body(p_ref, g_ref, out_ref):
    y = jnp.dot(p_ref[...], g_ref[...], preferred_element_type=jnp.float32)
    out_ref[...] = jax.nn.sigmoid(jax.nn.sigmoid(y))


def _tc_finish(partials, gmat):
    BF = 512
    return pl.pallas_call(
        _fin_body,
        grid=(PROWS // BF,),
        in_specs=[
            pl.BlockSpec((BF, D), lambda i: (i, 0)),
            pl.BlockSpec((D, 8), lambda i: (0, 0)),
        ],
        out_specs=pl.BlockSpec((BF, 8), lambda i: (i, 0)),
        out_shape=jax.ShapeDtypeStruct((PROWS, 8), jnp.float32),
    )(partials, gmat)


def kernel(features, edge_index, edge_types, c1_W, c1_b, c1_Wih, c1_Whh,
           c1_bih, c1_bhh, c2_W, c2_b, c2_Wih, c2_Whh, c2_bih, c2_bhh,
           W3, b3, W4, b4):
    src = edge_index[0]
    dst = edge_index[1]
    gidx = edge_types * N + src

    npad = EPAD - E
    pad_ids = jnp.arange(npad, dtype=jnp.int32)
    gidx_p = jnp.concatenate([gidx, pad_ids % 997])
    dst_p = jnp.concatenate([dst, N + (pad_ids % (NACC - N))])
    src_sp = jnp.concatenate([src, pad_ids % 997]).reshape(NW, NCH, CHUNK)
    dst_sp = jnp.concatenate(
        [dst, (pad_ids + 499) % 997]).reshape(NW, NCH, CHUNK)

    zeros_acc = jnp.zeros((NACC, D), jnp.float32)
    gmat = (jnp.arange(D, dtype=jnp.int32)[:, None] // L
            == jnp.arange(8, dtype=jnp.int32)[None, :]).astype(jnp.float32)

    def ggc(h, Wc, bc, Wih, Whh, bih, bhh):
        wt = jnp.transpose(Wc, (0, 2, 1))
        b3d = bc[:, None, :]
        wih_t = Wih.T
        whh_t = Whh.T
        bih2 = bih[None, :]
        bhh2 = bhh[None, :]
        for step in range(2):
            proj = _tc_proj(h, wt, b3d).reshape(K * N, D)
            partials = _seg_sum(proj, gidx_p, dst_p, zeros_acc)
            h = _tc_gru(partials, h, wih_t, whh_t, bih2, bhh2,
                        relu=(step == 1))
        return h

    h = ggc(features, c1_W, c1_b, c1_Wih, c1_Whh, c1_bih, c1_bhh)
    h = ggc(h, c2_W, c2_b, c2_Wih, c2_Whh, c2_bih, c2_bhh)
    x = _tc_mlp(h, W3.T, b3[None, :], W4.T, b4[None, :])
    part = _edge_score(x, src_sp, dst_sp)
    scores = _tc_finish(part, gmat).reshape(EPAD)
    return scores[:E]


# speedup vs baseline: 30.8973x; 1.1273x over previous
import functools

import jax
import jax.numpy as jnp
from jax import lax
from jax.experimental import pallas as pl
from jax.experimental.pallas import tpu as pltpu, tpu_sc as plsc

N, E, D, H, K = 10000, 320000, 128, 256, 4

NC, NS, L = 2, 16, 16
NW = NC * NS

CHUNK = 128
NCH = 81
EPW = NCH * CHUNK
EPAD = EPW * NW
NACC = 10112
RPT = NACC // NS
BP = 400
PROWS = EPAD // 8

_SC_PARAMS = pltpu.CompilerParams(needs_layout_passes=False)


def _mesh():
    return plsc.VectorSubcoreMesh(
        core_axis_name="c", subcore_axis_name="s",
        num_cores=NC, num_subcores=NS)


def _seg_sum_body(proj_hbm, gidx_hbm, dst_hbm, zero_hbm, out_hbm,
                  gi0, gi1, gi2, di0, di1, di2, r0, r1, r2, acc_sh,
                  sg0, sg1, sg2, si0, si1, si2, sd0, sd1, sd2,
                  sc0, sc1, sc2):
    gis = (gi0, gi1, gi2)
    dis = (di0, di1, di2)
    rows = (r0, r1, r2)
    sg = (sg0, sg1, sg2)
    si = (si0, si1, si2)
    sd = (sd0, sd1, sd2)
    ssc = (sc0, sc1, sc2)
    c = lax.axis_index("c")
    s = lax.axis_index("s")
    wid = s * NC + c
    base0 = wid * EPW
    pltpu.sync_copy(zero_hbm.at[pl.ds(s * RPT, RPT)],
                    acc_sh.at[pl.ds(s * RPT, RPT)])
    plsc.subcore_barrier()

    pltpu.sync_copy(gidx_hbm.at[pl.ds(base0, CHUNK)], gi0)
    pltpu.sync_copy(dst_hbm.at[pl.ds(base0, CHUNK)], di0)
    pltpu.async_copy(proj_hbm.at[gi0], r0, sg0)
    pltpu.async_copy(gidx_hbm.at[pl.ds(base0 + CHUNK, CHUNK)], gi1, si1)
    pltpu.async_copy(dst_hbm.at[pl.ds(base0 + CHUNK, CHUNK)], di1, sd1)

    def body(i, carry):
        for b in range(3):
            cc = 3 * i + b
            nx = (b + 1) % 3
            pv = (b + 2) % 3

            @pl.when(cc >= 1)
            def _():
                pltpu.make_async_copy(rows[pv], acc_sh.at[dis[pv]],
                                      ssc[pv]).wait()

            @pl.when(cc + 1 < NCH)
            def _():
                pltpu.make_async_copy(
                    gidx_hbm.at[pl.ds(base0 + (cc + 1) * CHUNK, CHUNK)],
                    gis[nx], si[nx]).wait()
                pltpu.make_async_copy(
                    dst_hbm.at[pl.ds(base0 + (cc + 1) * CHUNK, CHUNK)],
                    dis[nx], sd[nx]).wait()
                pltpu.async_copy(proj_hbm.at[gis[nx]], rows[nx], sg[nx])

            @pl.when(cc + 2 < NCH)
            def _():
                pltpu.async_copy(
                    gidx_hbm.at[pl.ds(base0 + (cc + 2) * CHUNK, CHUNK)],
                    gis[pv], si[pv])
                pltpu.async_copy(
                    dst_hbm.at[pl.ds(base0 + (cc + 2) * CHUNK, CHUNK)],
                    dis[pv], sd[pv])

            pltpu.make_async_copy(proj_hbm.at[gis[b]], rows[b], sg[b]).wait()
            pltpu.async_copy(rows[b], acc_sh.at[dis[b]], ssc[b], add=True)
        return carry

    lax.fori_loop(0, NCH // 3, body, 0)
    lsl = (NCH - 1) % 3
    pltpu.make_async_copy(rows[lsl], acc_sh.at[dis[lsl]], ssc[lsl]).wait()
    plsc.subcore_barrier()
    pltpu.sync_copy(acc_sh.at[pl.ds(s * RPT, RPT)],
                    out_hbm.at[c].at[pl.ds(s * RPT, RPT)])


@functools.cache
def _seg_sum_kernel():
    return pl.kernel(
        _seg_sum_body,
        out_type=jax.ShapeDtypeStruct((NC, NACC, D), jnp.float32),
        mesh=_mesh(),
        scratch_types=(
            [pltpu.VMEM((CHUNK,), jnp.int32) for _ in range(6)]
            + [pltpu.VMEM((CHUNK, D), jnp.float32) for _ in range(3)]
            + [pltpu.VMEM_SHARED((NACC, D), jnp.float32)]
            + [pltpu.SemaphoreType.DMA for _ in range(12)]
        ),
        compiler_params=_SC_PARAMS,
    )


def _seg_sum(proj, gidx_p, dst_p, zeros_acc):
    return _seg_sum_kernel()(proj, gidx_p, dst_p, zeros_acc)


def _edge_score_body(x_hbm, src_hbm, dst_hbm, out_hbm,
                     si_v, di_v, rs0, rs1, rs2, rd0, rd1, rd2,
                     p0, p1, p2,
                     sr0, sr1, sr2, sd0, sd1, sd2, so0, so1, so2):
    rsb = (rs0, rs1, rs2)
    rdb = (rd0, rd1, rd2)
    pvs = (p0, p1, p2)
    srs = (sr0, sr1, sr2)
    sds = (sd0, sd1, sd2)
    sos = (so0, so1, so2)
    c = lax.axis_index("c")
    s = lax.axis_index("s")
    wid = s * NC + c
    pltpu.sync_copy(src_hbm.at[wid], si_v)
    pltpu.sync_copy(dst_hbm.at[wid], di_v)
    pltpu.async_copy(x_hbm.at[si_v.at[0]], rs0, sr0)
    pltpu.async_copy(x_hbm.at[di_v.at[0]], rd0, sd0)
    pltpu.async_copy(x_hbm.at[si_v.at[1]], rs1, sr1)
    pltpu.async_copy(x_hbm.at[di_v.at[1]], rd1, sd1)

    def body(i, carry):
        for b in range(3):
            cc = 3 * i + b
            pv = (b + 2) % 3

            @pl.when(cc + 2 < NCH)
            def _():
                pltpu.async_copy(x_hbm.at[si_v.at[cc + 2]], rsb[pv], srs[pv])
                pltpu.async_copy(x_hbm.at[di_v.at[cc + 2]], rdb[pv], sds[pv])

            pltpu.make_async_copy(x_hbm.at[si_v.at[cc]], rsb[b], srs[b]).wait()
            pltpu.make_async_copy(x_hbm.at[di_v.at[cc]], rdb[b], sds[b]).wait()

            @pl.when(cc >= 3)
            def _():
                pltpu.make_async_copy(
                    pvs[b], out_hbm.at[pl.ds(0, CHUNK // 8)], sos[b]).wait()

            rs, rd, p_v = rsb[b], rdb[b], pvs[b]

            def gbody(g, carry2):
                for j in range(8):
                    e = g * 8 + j
                    acc = rs[e, pl.ds(0, L)] * rd[e, pl.ds(0, L)]
                    for fc in range(1, 8):
                        acc = acc + (rs[e, pl.ds(fc * L, L)]
                                     * rd[e, pl.ds(fc * L, L)])
                    p_v[g, pl.ds(j * L, L)] = acc
                return carry2

            lax.fori_loop(0, CHUNK // 8, gbody, 0)
            base_row = wid * (EPW // 8) + cc * (CHUNK // 8)
            pltpu.async_copy(p_v, out_hbm.at[pl.ds(base_row, CHUNK // 8)],
                             sos[b])
        return carry

    lax.fori_loop(0, NCH // 3, body, 0)
    for b in range(3):
        pltpu.make_async_copy(pvs[b], out_hbm.at[pl.ds(0, CHUNK // 8)],
                              sos[b]).wait()


@functools.cache
def _edge_score_kernel():
    return pl.kernel(
        _edge_score_body,
        out_type=jax.ShapeDtypeStruct((PROWS, D), jnp.float32),
        mesh=_mesh(),
        scratch_types=(
            [pltpu.VMEM((NCH, CHUNK), jnp.int32) for _ in range(2)]
            + [pltpu.VMEM((CHUNK, D), jnp.float32) for _ in range(6)]
            + [pltpu.VMEM((CHUNK // 8, D), jnp.float32) for _ in range(3)]
            + [pltpu.SemaphoreType.DMA for _ in range(9)]
        ),
        compiler_params=_SC_PARAMS,
    )


def _edge_score(x, src_sp, dst_sp):
    return _edge_score_kernel()(x, src_sp, dst_sp)


def _proj_body(h_ref, wt_ref, b_ref, out_ref):
    h = h_ref[...]
    for k in range(K):
        out_ref[k] = (jnp.dot(h, wt_ref[k], preferred_element_type=jnp.float32)
                      + b_ref[k])


def _tc_proj(h, wt, b3d):
    return pl.pallas_call(
        _proj_body,
        grid=(N // BP,),
        in_specs=[
            pl.BlockSpec((BP, D), lambda i: (i, 0)),
            pl.BlockSpec((K, D, D), lambda i: (0, 0, 0)),
            pl.BlockSpec((K, 1, D), lambda i: (0, 0, 0)),
        ],
        out_specs=pl.BlockSpec((K, BP, D), lambda i: (0, i, 0)),
        out_shape=jax.ShapeDtypeStruct((K, N, D), jnp.float32),
    )(h, wt, b3d)


def _gru_core(p_ref, h_ref, wih_ref, whh_ref, bih_ref, bhh_ref, relu):
    a = p_ref[0] + p_ref[1]
    h = h_ref[...]
    gi = jnp.dot(a, wih_ref[...], preferred_element_type=jnp.float32) + bih_ref[...]
    gh = jnp.dot(h, whh_ref[...], preferred_element_type=jnp.float32) + bhh_ref[...]
    r = jax.nn.sigmoid(gi[:, :D] + gh[:, :D])
    z = jax.nn.sigmoid(gi[:, D:2 * D] + gh[:, D:2 * D])
    n = jnp.tanh(gi[:, 2 * D:] + r * gh[:, 2 * D:])
    hn = (1.0 - z) * n + z * h
    if relu:
        hn = jnp.maximum(hn, 0.0)
    return hn


def _gru_proj_body(relu, p_ref, h_ref, wih_ref, whh_ref, bih_ref, bhh_ref,
                   wt_ref, b_ref, hn_ref, proj_ref):
    hn = _gru_core(p_ref, h_ref, wih_ref, whh_ref, bih_ref, bhh_ref, relu)
    hn_ref[...] = hn
    for k in range(K):
        proj_ref[k] = (jnp.dot(hn, wt_ref[k], preferred_element_type=jnp.float32)
                       + b_ref[k])


def _tc_gru_proj(partials, h, wih_t, whh_t, bih2, bhh2, wt, b3d, relu):
    return pl.pallas_call(
        functools.partial(_gru_proj_body, relu),
        grid=(N // BP,),
        in_specs=[
            pl.BlockSpec((NC, BP, D), lambda i: (0, i, 0)),
            pl.BlockSpec((BP, D), lambda i: (i, 0)),
            pl.BlockSpec((D, 3 * D), lambda i: (0, 0)),
            pl.BlockSpec((D, 3 * D), lambda i: (0, 0)),
            pl.BlockSpec((1, 3 * D), lambda i: (0, 0)),
            pl.BlockSpec((1, 3 * D), lambda i: (0, 0)),
            pl.BlockSpec((K, D, D), lambda i: (0, 0, 0)),
            pl.BlockSpec((K, 1, D), lambda i: (0, 0, 0)),
        ],
        out_specs=[
            pl.BlockSpec((BP, D), lambda i: (i, 0)),
            pl.BlockSpec((K, BP, D), lambda i: (0, i, 0)),
        ],
        out_shape=[
            jax.ShapeDtypeStruct((N, D), jnp.float32),
            jax.ShapeDtypeStruct((K, N, D), jnp.float32),
        ],
    )(partials, h, wih_t, whh_t, bih2, bhh2, wt, b3d)


def _gru_body(relu, p_ref, h_ref, wih_ref, whh_ref, bih_ref, bhh_ref, out_ref):
    out_ref[...] = _gru_core(p_ref, h_ref, wih_ref, whh_ref, bih_ref,
                             bhh_ref, relu)


def _tc_gru(partials, h, wih_t, whh_t, bih2, bhh2, relu):
    return pl.pallas_call(
        functools.partial(_gru_body, relu),
        grid=(N // BP,),
        in_specs=[
            pl.BlockSpec((NC, BP, D), lambda i: (0, i, 0)),
            pl.BlockSpec((BP, D), lambda i: (i, 0)),
            pl.BlockSpec((D, 3 * D), lambda i: (0, 0)),
            pl.BlockSpec((D, 3 * D), lambda i: (0, 0)),
            pl.BlockSpec((1, 3 * D), lambda i: (0, 0)),
            pl.BlockSpec((1, 3 * D), lambda i: (0, 0)),
        ],
        out_specs=pl.BlockSpec((BP, D), lambda i: (i, 0)),
        out_shape=jax.ShapeDtypeStruct((N, D), jnp.float32),
    )(partials, h, wih_t, whh_t, bih2, bhh2)


def _mlp_body(x_ref, w3_ref, b3_ref, w4_ref, b4_ref, out_ref):
    x = x_ref[...]
    y = jnp.maximum(
        jnp.dot(x, w3_ref[...], preferred_element_type=jnp.float32) + b3_ref[...],
        0.0)
    z = jnp.maximum(
        jnp.dot(y, w4_ref[...], preferred_element_type=jnp.float32) + b4_ref[...],
        0.0)
    out_ref[...] = z


def _tc_mlp(x, w3t, b32, w4t, b42):
    return pl.pallas_call(
        _mlp_body,
        grid=(N // BP,),
        in_specs=[
            pl.BlockSpec((BP, D), lambda i: (i, 0)),
            pl.BlockSpec((D, H), lambda i: (0, 0)),
            pl.BlockSpec((1, H), lambda i: (0, 0)),
            pl.BlockSpec((H, D), lambda i: (0, 0)),
            pl.BlockSpec((1, D), lambda i: (0, 0)),
        ],
        out_specs=pl.BlockSpec((BP, D), lambda i: (i, 0)),
        out_shape=jax.ShapeDtypeStruct((N, D), jnp.float32),
    )(x, w3t, b32, w4t, b42)


def _fin_body(p_ref, g_ref, out_ref):
    y = jnp.dot(p_ref[...], g_ref[...], preferred_element_type=jnp.float32)
    out_ref[...] = jax.nn.sigmoid(jax.nn.sigmoid(y))


def _tc_finish(partials, gmat):
    BF = 512
    return pl.pallas_call(
        _fin_body,
        grid=(PROWS // BF,),
        in_specs=[
            pl.BlockSpec((BF, D), lambda i: (i, 0)),
            pl.BlockSpec((D, 8), lambda i: (0, 0)),
        ],
        out_specs=pl.BlockSpec((BF, 8), lambda i: (i, 0)),
        out_shape=jax.ShapeDtypeStruct((PROWS, 8), jnp.float32),
    )(partials, gmat)


def kernel(features, edge_index, edge_types, c1_W, c1_b, c1_Wih, c1_Whh,
           c1_bih, c1_bhh, c2_W, c2_b, c2_Wih, c2_Whh, c2_bih, c2_bhh,
           W3, b3, W4, b4):
    src = edge_index[0]
    dst = edge_index[1]
    gidx = edge_types * N + src

    npad = EPAD - E
    pad_ids = jnp.arange(npad, dtype=jnp.int32)
    gidx_p = jnp.concatenate([gidx, pad_ids % 997])
    dst_p = jnp.concatenate([dst, N + (pad_ids % (NACC - N))])
    src_sp = jnp.concatenate([src, pad_ids % 997]).reshape(NW, NCH, CHUNK)
    dst_sp = jnp.concatenate(
        [dst, (pad_ids + 499) % 997]).reshape(NW, NCH, CHUNK)

    zeros_acc = jnp.zeros((NACC, D), jnp.float32)
    gmat = (jnp.arange(D, dtype=jnp.int32)[:, None] // L
            == jnp.arange(8, dtype=jnp.int32)[None, :]).astype(jnp.float32)

    wt1 = jnp.transpose(c1_W, (0, 2, 1))
    b1_3d = c1_b[:, None, :]
    wt2 = jnp.transpose(c2_W, (0, 2, 1))
    b2_3d = c2_b[:, None, :]
    g1 = (c1_Wih.T, c1_Whh.T, c1_bih[None, :], c1_bhh[None, :])
    g2 = (c2_Wih.T, c2_Whh.T, c2_bih[None, :], c2_bhh[None, :])

    h = features
    proj = _tc_proj(h, wt1, b1_3d)
    steps = [
        (g1, False, (wt1, b1_3d)),
        (g1, True, (wt2, b2_3d)),
        (g2, False, (wt2, b2_3d)),
        (g2, True, None),
    ]
    for gw, relu, nxt in steps:
        partials = _seg_sum(proj.reshape(K * N, D), gidx_p, dst_p, zeros_acc)
        if nxt is None:
            h = _tc_gru(partials, h, *gw, relu)
        else:
            h, proj = _tc_gru_proj(partials, h, *gw, *nxt, relu)

    x = _tc_mlp(h, W3.T, b3[None, :], W4.T, b4[None, :])
    part = _edge_score(x, src_sp, dst_sp)
    scores = _tc_finish(part, gmat).reshape(EPAD)
    return scores[:E]


# speedup vs baseline: 32.5100x; 1.0522x over previous
import functools

import jax
import jax.numpy as jnp
from jax import lax
from jax.experimental import pallas as pl
from jax.experimental.pallas import tpu as pltpu, tpu_sc as plsc

N, E, D, H, K = 10000, 320000, 128, 256, 4

NC, NS, L = 2, 16, 16
NW = NC * NS

CHUNK = 128
NCH = 81
EPW = NCH * CHUNK
EPAD = EPW * NW
NACC = 10112
RPT = NACC // NS
BP = 400
PROWS = EPAD // 8

_SC_PARAMS = pltpu.CompilerParams(needs_layout_passes=False)


def _mesh():
    return plsc.VectorSubcoreMesh(
        core_axis_name="c", subcore_axis_name="s",
        num_cores=NC, num_subcores=NS)


def _seg_sum_body(proj_hbm, idx_hbm, zero_hbm, out_hbm,
                  ib0, ib1, ib2, r0, r1, r2, acc_sh,
                  sg0, sg1, sg2, si0, si1, si2, sc0, sc1, sc2):
    ibs = (ib0, ib1, ib2)
    rows = (r0, r1, r2)
    sg = (sg0, sg1, sg2)
    si = (si0, si1, si2)
    ssc = (sc0, sc1, sc2)
    c = lax.axis_index("c")
    s = lax.axis_index("s")
    wid = s * NC + c
    pltpu.sync_copy(zero_hbm.at[pl.ds(s * RPT, RPT)],
                    acc_sh.at[pl.ds(s * RPT, RPT)])
    plsc.subcore_barrier()

    pltpu.sync_copy(idx_hbm.at[wid, 0], ib0)
    pltpu.async_copy(proj_hbm.at[ib0.at[0]], r0, sg0)
    pltpu.async_copy(idx_hbm.at[wid, 1], ib1, si1)

    def body(i, carry):
        for b in range(3):
            cc = 3 * i + b
            nx = (b + 1) % 3
            pv = (b + 2) % 3

            @pl.when(cc >= 1)
            def _():
                pltpu.make_async_copy(rows[pv], acc_sh.at[ibs[pv].at[1]],
                                      ssc[pv]).wait()

            @pl.when(cc + 1 < NCH)
            def _():
                pltpu.make_async_copy(idx_hbm.at[wid, cc + 1],
                                      ibs[nx], si[nx]).wait()
                pltpu.async_copy(proj_hbm.at[ibs[nx].at[0]], rows[nx], sg[nx])

            @pl.when(cc + 2 < NCH)
            def _():
                pltpu.async_copy(idx_hbm.at[wid, cc + 2], ibs[pv], si[pv])

            pltpu.make_async_copy(proj_hbm.at[ibs[b].at[0]], rows[b],
                                  sg[b]).wait()
            pltpu.async_copy(rows[b], acc_sh.at[ibs[b].at[1]], ssc[b],
                             add=True)
        return carry

    lax.fori_loop(0, NCH // 3, body, 0)
    lsl = (NCH - 1) % 3
    pltpu.make_async_copy(rows[lsl], acc_sh.at[ibs[lsl].at[1]],
                          ssc[lsl]).wait()
    plsc.subcore_barrier()
    pltpu.sync_copy(acc_sh.at[pl.ds(s * RPT, RPT)],
                    out_hbm.at[c].at[pl.ds(s * RPT, RPT)])


@functools.cache
def _seg_sum_kernel():
    return pl.kernel(
        _seg_sum_body,
        out_type=jax.ShapeDtypeStruct((NC, NACC, D), jnp.float32),
        mesh=_mesh(),
        scratch_types=(
            [pltpu.VMEM((2, CHUNK), jnp.int32) for _ in range(3)]
            + [pltpu.VMEM((CHUNK, D), jnp.float32) for _ in range(3)]
            + [pltpu.VMEM_SHARED((NACC, D), jnp.float32)]
            + [pltpu.SemaphoreType.DMA for _ in range(9)]
        ),
        compiler_params=_SC_PARAMS,
    )


def _seg_sum(proj, segidx, zeros_acc):
    return _seg_sum_kernel()(proj, segidx, zeros_acc)


def _edge_score_body(x_hbm, idx_hbm, out_hbm,
                     si_v, rr0, rr1, rr2, p0, p1, p2,
                     sr0, sr1, sr2, so0, so1, so2):
    rrb = (rr0, rr1, rr2)
    pvs = (p0, p1, p2)
    srs = (sr0, sr1, sr2)
    sos = (so0, so1, so2)
    c = lax.axis_index("c")
    s = lax.axis_index("s")
    wid = s * NC + c
    pltpu.sync_copy(idx_hbm.at[wid], si_v)

    def _launch(cc, sl):
        pltpu.async_copy(x_hbm.at[si_v.at[cc].at[0]], rrb[sl].at[0], srs[sl])
        pltpu.async_copy(x_hbm.at[si_v.at[cc].at[1]], rrb[sl].at[1], srs[sl])

    def _wait(cc, sl):
        pltpu.make_async_copy(x_hbm.at[si_v.at[cc].at[0]], rrb[sl].at[0],
                              srs[sl]).wait()
        pltpu.make_async_copy(x_hbm.at[si_v.at[cc].at[1]], rrb[sl].at[1],
                              srs[sl]).wait()

    _launch(0, 0)
    _launch(1, 1)

    def body(i, carry):
        for b in range(3):
            cc = 3 * i + b
            pv = (b + 2) % 3

            @pl.when(cc + 2 < NCH)
            def _():
                _launch(cc + 2, pv)

            _wait(cc, b)

            @pl.when(cc >= 3)
            def _():
                pltpu.make_async_copy(
                    pvs[b], out_hbm.at[pl.ds(0, CHUNK // 8)], sos[b]).wait()

            rs, rd, p_v = rrb[b].at[0], rrb[b].at[1], pvs[b]

            def gbody(g, carry2):
                for j in range(8):
                    e = g * 8 + j
                    acc = rs[e, pl.ds(0, L)] * rd[e, pl.ds(0, L)]
                    for fc in range(1, 8):
                        acc = acc + (rs[e, pl.ds(fc * L, L)]
                                     * rd[e, pl.ds(fc * L, L)])
                    p_v[g, pl.ds(j * L, L)] = acc
                return carry2

            lax.fori_loop(0, CHUNK // 8, gbody, 0)
            base_row = wid * (EPW // 8) + cc * (CHUNK // 8)
            pltpu.async_copy(p_v, out_hbm.at[pl.ds(base_row, CHUNK // 8)],
                             sos[b])
        return carry

    lax.fori_loop(0, NCH // 3, body, 0)
    for b in range(3):
        pltpu.make_async_copy(pvs[b], out_hbm.at[pl.ds(0, CHUNK // 8)],
                              sos[b]).wait()


@functools.cache
def _edge_score_kernel():
    return pl.kernel(
        _edge_score_body,
        out_type=jax.ShapeDtypeStruct((PROWS, D), jnp.float32),
        mesh=_mesh(),
        scratch_types=(
            [pltpu.VMEM((NCH, 2, CHUNK), jnp.int32)]
            + [pltpu.VMEM((2, CHUNK, D), jnp.float32) for _ in range(3)]
            + [pltpu.VMEM((CHUNK // 8, D), jnp.float32) for _ in range(3)]
            + [pltpu.SemaphoreType.DMA for _ in range(6)]
        ),
        compiler_params=_SC_PARAMS,
    )


def _edge_score(x, scidx):
    return _edge_score_kernel()(x, scidx)


def _proj_body(h_ref, wt_ref, b_ref, out_ref):
    h = h_ref[...]
    for k in range(K):
        out_ref[k] = (jnp.dot(h, wt_ref[k], preferred_element_type=jnp.float32)
                      + b_ref[k])


def _tc_proj(h, wt, b3d):
    return pl.pallas_call(
        _proj_body,
        grid=(N // BP,),
        in_specs=[
            pl.BlockSpec((BP, D), lambda i: (i, 0)),
            pl.BlockSpec((K, D, D), lambda i: (0, 0, 0)),
            pl.BlockSpec((K, 1, D), lambda i: (0, 0, 0)),
        ],
        out_specs=pl.BlockSpec((K, BP, D), lambda i: (0, i, 0)),
        out_shape=jax.ShapeDtypeStruct((K, N, D), jnp.float32),
    )(h, wt, b3d)


def _gru_core(p_ref, h_ref, wih_ref, whh_ref, bih_ref, bhh_ref, relu):
    a = p_ref[0] + p_ref[1]
    h = h_ref[...]
    gi = jnp.dot(a, wih_ref[...], preferred_element_type=jnp.float32) + bih_ref[...]
    gh = jnp.dot(h, whh_ref[...], preferred_element_type=jnp.float32) + bhh_ref[...]
    r = jax.nn.sigmoid(gi[:, :D] + gh[:, :D])
    z = jax.nn.sigmoid(gi[:, D:2 * D] + gh[:, D:2 * D])
    n = jnp.tanh(gi[:, 2 * D:] + r * gh[:, 2 * D:])
    hn = (1.0 - z) * n + z * h
    if relu:
        hn = jnp.maximum(hn, 0.0)
    return hn


def _gru_proj_body(relu, p_ref, h_ref, wih_ref, whh_ref, bih_ref, bhh_ref,
                   wt_ref, b_ref, hn_ref, proj_ref):
    hn = _gru_core(p_ref, h_ref, wih_ref, whh_ref, bih_ref, bhh_ref, relu)
    hn_ref[...] = hn
    for k in range(K):
        proj_ref[k] = (jnp.dot(hn, wt_ref[k], preferred_element_type=jnp.float32)
                       + b_ref[k])


def _tc_gru_proj(partials, h, wih_t, whh_t, bih2, bhh2, wt, b3d, relu):
    return pl.pallas_call(
        functools.partial(_gru_proj_body, relu),
        grid=(N // BP,),
        in_specs=[
            pl.BlockSpec((NC, BP, D), lambda i: (0, i, 0)),
            pl.BlockSpec((BP, D), lambda i: (i, 0)),
            pl.BlockSpec((D, 3 * D), lambda i: (0, 0)),
            pl.BlockSpec((D, 3 * D), lambda i: (0, 0)),
            pl.BlockSpec((1, 3 * D), lambda i: (0, 0)),
            pl.BlockSpec((1, 3 * D), lambda i: (0, 0)),
            pl.BlockSpec((K, D, D), lambda i: (0, 0, 0)),
            pl.BlockSpec((K, 1, D), lambda i: (0, 0, 0)),
        ],
        out_specs=[
            pl.BlockSpec((BP, D), lambda i: (i, 0)),
            pl.BlockSpec((K, BP, D), lambda i: (0, i, 0)),
        ],
        out_shape=[
            jax.ShapeDtypeStruct((N, D), jnp.float32),
            jax.ShapeDtypeStruct((K, N, D), jnp.float32),
        ],
    )(partials, h, wih_t, whh_t, bih2, bhh2, wt, b3d)


def _gru_mlp_body(p_ref, h_ref, wih_ref, whh_ref, bih_ref, bhh_ref,
                  w3_ref, b3_ref, w4_ref, b4_ref, out_ref):
    hn = _gru_core(p_ref, h_ref, wih_ref, whh_ref, bih_ref, bhh_ref, True)
    y = jnp.maximum(
        jnp.dot(hn, w3_ref[...], preferred_element_type=jnp.float32) + b3_ref[...],
        0.0)
    z = jnp.maximum(
        jnp.dot(y, w4_ref[...], preferred_element_type=jnp.float32) + b4_ref[...],
        0.0)
    out_ref[...] = z


def _tc_gru_mlp(partials, h, wih_t, whh_t, bih2, bhh2, w3t, b32, w4t, b42):
    return pl.pallas_call(
        _gru_mlp_body,
        grid=(N // BP,),
        in_specs=[
            pl.BlockSpec((NC, BP, D), lambda i: (0, i, 0)),
            pl.BlockSpec((BP, D), lambda i: (i, 0)),
            pl.BlockSpec((D, 3 * D), lambda i: (0, 0)),
            pl.BlockSpec((D, 3 * D), lambda i: (0, 0)),
            pl.BlockSpec((1, 3 * D), lambda i: (0, 0)),
            pl.BlockSpec((1, 3 * D), lambda i: (0, 0)),
            pl.BlockSpec((D, H), lambda i: (0, 0)),
            pl.BlockSpec((1, H), lambda i: (0, 0)),
            pl.BlockSpec((H, D), lambda i: (0, 0)),
            pl.BlockSpec((1, D), lambda i: (0, 0)),
        ],
        out_specs=pl.BlockSpec((BP, D), lambda i: (i, 0)),
        out_shape=jax.ShapeDtypeStruct((N, D), jnp.float32),
    )(partials, h, wih_t, whh_t, bih2, bhh2, w3t, b32, w4t, b42)


def _fin_body(p_ref, g_ref, out_ref):
    y = jnp.dot(p_ref[...], g_ref[...], preferred_element_type=jnp.float32)
    out_ref[...] = jax.nn.sigmoid(jax.nn.sigmoid(y))


def _tc_finish(partials, gmat):
    BF = 1728
    return pl.pallas_call(
        _fin_body,
        grid=(PROWS // BF,),
        in_specs=[
            pl.BlockSpec((BF, D), lambda i: (i, 0)),
            pl.BlockSpec((D, 8), lambda i: (0, 0)),
        ],
        out_specs=pl.BlockSpec((BF, 8), lambda i: (i, 0)),
        out_shape=jax.ShapeDtypeStruct((PROWS, 8), jnp.float32),
    )(partials, gmat)


def kernel(features, edge_index, edge_types, c1_W, c1_b, c1_Wih, c1_Whh,
           c1_bih, c1_bhh, c2_W, c2_b, c2_Wih, c2_Whh, c2_bih, c2_bhh,
           W3, b3, W4, b4):
    src = edge_index[0]
    dst = edge_index[1]
    gidx = edge_types * N + src

    npad = EPAD - E
    pad_ids = jnp.arange(npad, dtype=jnp.int32)
    gidx_p = jnp.concatenate([gidx, pad_ids % 997]).reshape(NW, NCH, CHUNK)
    dst_p = jnp.concatenate(
        [dst, N + (pad_ids % (NACC - N))]).reshape(NW, NCH, CHUNK)
    segidx = jnp.stack([gidx_p, dst_p], axis=2)
    src_sp = jnp.concatenate([src, pad_ids % 997]).reshape(NW, NCH, CHUNK)
    dst_sp = jnp.concatenate(
        [dst, (pad_ids + 499) % 997]).reshape(NW, NCH, CHUNK)
    scidx = jnp.stack([src_sp, dst_sp], axis=2)

    zeros_acc = jnp.zeros((NACC, D), jnp.float32)
    gmat = (jnp.arange(D, dtype=jnp.int32)[:, None] // L
            == jnp.arange(8, dtype=jnp.int32)[None, :]).astype(jnp.float32)

    wt1 = jnp.transpose(c1_W, (0, 2, 1))
    b1_3d = c1_b[:, None, :]
    wt2 = jnp.transpose(c2_W, (0, 2, 1))
    b2_3d = c2_b[:, None, :]
    g1 = (c1_Wih.T, c1_Whh.T, c1_bih[None, :], c1_bhh[None, :])
    g2 = (c2_Wih.T, c2_Whh.T, c2_bih[None, :], c2_bhh[None, :])

    h = features
    proj = _tc_proj(h, wt1, b1_3d)
    steps = [
        (g1, False, (wt1, b1_3d)),
        (g1, True, (wt2, b2_3d)),
        (g2, False, (wt2, b2_3d)),
        (g2, True, None),
    ]
    for gw, relu, nxt in steps:
        partials = _seg_sum(proj.reshape(K * N, D), segidx, zeros_acc)
        if nxt is None:
            x = _tc_gru_mlp(partials, h, *gw,
                            W3.T, b3[None, :], W4.T, b4[None, :])
        else:
            h, proj = _tc_gru_proj(partials, h, *gw, *nxt, relu)

    part = _edge_score(x, scidx)
    scores = _tc_finish(part, gmat).reshape(EPAD)
    return scores[:E]


# speedup vs baseline: 32.8989x; 1.0120x over previous
import functools

import jax
import jax.numpy as jnp
from jax import lax
from jax.experimental import pallas as pl
from jax.experimental.pallas import tpu as pltpu, tpu_sc as plsc

N, E, D, H, K = 10000, 320000, 128, 256, 4

NC, NS, L = 2, 16, 16
NW = NC * NS

CHUNK = 128
NCH = 81
EPW = NCH * CHUNK
EPAD = EPW * NW
NACC = 10112
RPT = NACC // NS
BP = 400
PROWS = EPAD // 8

_SC_PARAMS = pltpu.CompilerParams(needs_layout_passes=False)


def _mesh():
    return plsc.VectorSubcoreMesh(
        core_axis_name="c", subcore_axis_name="s",
        num_cores=NC, num_subcores=NS)


def _seg_sum_body(proj_hbm, idx_hbm, zero_hbm, out_hbm,
                  ib0, ib1, ib2, r0, r1, r2, acc_sh,
                  sg0, sg1, sg2, si0, si1, si2, sc0, sc1, sc2):
    ibs = (ib0, ib1, ib2)
    rows = (r0, r1, r2)
    sg = (sg0, sg1, sg2)
    si = (si0, si1, si2)
    ssc = (sc0, sc1, sc2)
    c = lax.axis_index("c")
    s = lax.axis_index("s")
    wid = s * NC + c
    pltpu.sync_copy(zero_hbm.at[pl.ds(s * RPT, RPT)],
                    acc_sh.at[pl.ds(s * RPT, RPT)])
    plsc.subcore_barrier()

    pltpu.sync_copy(idx_hbm.at[wid, 0], ib0)
    pltpu.async_copy(proj_hbm.at[ib0.at[0]], r0, sg0)
    pltpu.async_copy(idx_hbm.at[wid, 1], ib1, si1)

    def body(i, carry):
        for b in range(3):
            cc = 3 * i + b
            nx = (b + 1) % 3
            pv = (b + 2) % 3

            @pl.when(cc + 1 < NCH)
            def _():
                pltpu.make_async_copy(idx_hbm.at[wid, cc + 1],
                                      ibs[nx], si[nx]).wait()
                pltpu.async_copy(proj_hbm.at[ibs[nx].at[0]], rows[nx], sg[nx])

            @pl.when(cc >= 1)
            def _():
                pltpu.make_async_copy(rows[pv], acc_sh.at[ibs[pv].at[1]],
                                      ssc[pv]).wait()

            @pl.when(cc + 2 < NCH)
            def _():
                pltpu.async_copy(idx_hbm.at[wid, cc + 2], ibs[pv], si[pv])

            pltpu.make_async_copy(proj_hbm.at[ibs[b].at[0]], rows[b],
                                  sg[b]).wait()
            pltpu.async_copy(rows[b], acc_sh.at[ibs[b].at[1]], ssc[b],
                             add=True)
        return carry

    lax.fori_loop(0, NCH // 3, body, 0)
    lsl = (NCH - 1) % 3
    pltpu.make_async_copy(rows[lsl], acc_sh.at[ibs[lsl].at[1]],
                          ssc[lsl]).wait()
    plsc.subcore_barrier()
    pltpu.sync_copy(acc_sh.at[pl.ds(s * RPT, RPT)],
                    out_hbm.at[c].at[pl.ds(s * RPT, RPT)])


@functools.cache
def _seg_sum_kernel():
    return pl.kernel(
        _seg_sum_body,
        out_type=jax.ShapeDtypeStruct((NC, NACC, D), jnp.float32),
        mesh=_mesh(),
        scratch_types=(
            [pltpu.VMEM((2, CHUNK), jnp.int32) for _ in range(3)]
            + [pltpu.VMEM((CHUNK, D), jnp.float32) for _ in range(3)]
            + [pltpu.VMEM_SHARED((NACC, D), jnp.float32)]
            + [pltpu.SemaphoreType.DMA for _ in range(9)]
        ),
        compiler_params=_SC_PARAMS,
    )


def _seg_sum(proj, segidx, zeros_acc):
    return _seg_sum_kernel()(proj, segidx, zeros_acc)


def _edge_score_body(x_hbm, idx_hbm, out_hbm,
                     si_v, rr0, rr1, rr2, p0, p1, p2,
                     sr0, sr1, sr2, so0, so1, so2):
    rrb = (rr0, rr1, rr2)
    pvs = (p0, p1, p2)
    srs = (sr0, sr1, sr2)
    sos = (so0, so1, so2)
    c = lax.axis_index("c")
    s = lax.axis_index("s")
    wid = s * NC + c
    pltpu.sync_copy(idx_hbm.at[wid], si_v)

    def _launch(cc, sl):
        pltpu.async_copy(x_hbm.at[si_v.at[cc].at[0]], rrb[sl].at[0], srs[sl])
        pltpu.async_copy(x_hbm.at[si_v.at[cc].at[1]], rrb[sl].at[1], srs[sl])

    def _wait(cc, sl):
        pltpu.make_async_copy(x_hbm.at[si_v.at[cc].at[0]], rrb[sl].at[0],
                              srs[sl]).wait()
        pltpu.make_async_copy(x_hbm.at[si_v.at[cc].at[1]], rrb[sl].at[1],
                              srs[sl]).wait()

    _launch(0, 0)
    _launch(1, 1)

    def body(i, carry):
        for b in range(3):
            cc = 3 * i + b
            pv = (b + 2) % 3

            @pl.when(cc + 2 < NCH)
            def _():
                _launch(cc + 2, pv)

            _wait(cc, b)

            @pl.when(cc >= 3)
            def _():
                pltpu.make_async_copy(
                    pvs[b], out_hbm.at[pl.ds(0, CHUNK // 8)], sos[b]).wait()

            rs, rd, p_v = rrb[b].at[0], rrb[b].at[1], pvs[b]

            def gbody(g, carry2):
                for j in range(8):
                    e = g * 8 + j
                    acc = rs[e, pl.ds(0, L)] * rd[e, pl.ds(0, L)]
                    for fc in range(1, 8):
                        acc = acc + (rs[e, pl.ds(fc * L, L)]
                                     * rd[e, pl.ds(fc * L, L)])
                    p_v[g, pl.ds(j * L, L)] = acc
                return carry2

            lax.fori_loop(0, CHUNK // 8, gbody, 0)
            base_row = wid * (EPW // 8) + cc * (CHUNK // 8)
            pltpu.async_copy(p_v, out_hbm.at[pl.ds(base_row, CHUNK // 8)],
                             sos[b])
        return carry

    lax.fori_loop(0, NCH // 3, body, 0)
    for b in range(3):
        pltpu.make_async_copy(pvs[b], out_hbm.at[pl.ds(0, CHUNK // 8)],
                              sos[b]).wait()


@functools.cache
def _edge_score_kernel():
    return pl.kernel(
        _edge_score_body,
        out_type=jax.ShapeDtypeStruct((PROWS, D), jnp.float32),
        mesh=_mesh(),
        scratch_types=(
            [pltpu.VMEM((NCH, 2, CHUNK), jnp.int32)]
            + [pltpu.VMEM((2, CHUNK, D), jnp.float32) for _ in range(3)]
            + [pltpu.VMEM((CHUNK // 8, D), jnp.float32) for _ in range(3)]
            + [pltpu.SemaphoreType.DMA for _ in range(6)]
        ),
        compiler_params=_SC_PARAMS,
    )


def _edge_score(x, scidx):
    return _edge_score_kernel()(x, scidx)


def _proj_body(h_ref, wt_ref, b_ref, out_ref):
    h = h_ref[...]
    for k in range(K):
        out_ref[k] = (jnp.dot(h, wt_ref[k], preferred_element_type=jnp.float32)
                      + b_ref[k])


def _tc_proj(h, wt, b3d):
    return pl.pallas_call(
        _proj_body,
        grid=(N // BP,),
        in_specs=[
            pl.BlockSpec((BP, D), lambda i: (i, 0)),
            pl.BlockSpec((K, D, D), lambda i: (0, 0, 0)),
            pl.BlockSpec((K, 1, D), lambda i: (0, 0, 0)),
        ],
        out_specs=pl.BlockSpec((K, BP, D), lambda i: (0, i, 0)),
        out_shape=jax.ShapeDtypeStruct((K, N, D), jnp.float32),
    )(h, wt, b3d)


def _gru_core(p_ref, h_ref, wih_ref, whh_ref, bih_ref, bhh_ref, relu):
    a = p_ref[0] + p_ref[1]
    h = h_ref[...]
    gi = jnp.dot(a, wih_ref[...], preferred_element_type=jnp.float32) + bih_ref[...]
    gh = jnp.dot(h, whh_ref[...], preferred_element_type=jnp.float32) + bhh_ref[...]
    r = jax.nn.sigmoid(gi[:, :D] + gh[:, :D])
    z = jax.nn.sigmoid(gi[:, D:2 * D] + gh[:, D:2 * D])
    n = jnp.tanh(gi[:, 2 * D:] + r * gh[:, 2 * D:])
    hn = (1.0 - z) * n + z * h
    if relu:
        hn = jnp.maximum(hn, 0.0)
    return hn


def _gru_proj_body(relu, p_ref, h_ref, wih_ref, whh_ref, bih_ref, bhh_ref,
                   wt_ref, b_ref, hn_ref, proj_ref):
    hn = _gru_core(p_ref, h_ref, wih_ref, whh_ref, bih_ref, bhh_ref, relu)
    hn_ref[...] = hn
    for k in range(K):
        proj_ref[k] = (jnp.dot(hn, wt_ref[k], preferred_element_type=jnp.float32)
                       + b_ref[k])


def _tc_gru_proj(partials, h, wih_t, whh_t, bih2, bhh2, wt, b3d, relu):
    return pl.pallas_call(
        functools.partial(_gru_proj_body, relu),
        grid=(N // BP,),
        in_specs=[
            pl.BlockSpec((NC, BP, D), lambda i: (0, i, 0)),
            pl.BlockSpec((BP, D), lambda i: (i, 0)),
            pl.BlockSpec((D, 3 * D), lambda i: (0, 0)),
            pl.BlockSpec((D, 3 * D), lambda i: (0, 0)),
            pl.BlockSpec((1, 3 * D), lambda i: (0, 0)),
            pl.BlockSpec((1, 3 * D), lambda i: (0, 0)),
            pl.BlockSpec((K, D, D), lambda i: (0, 0, 0)),
            pl.BlockSpec((K, 1, D), lambda i: (0, 0, 0)),
        ],
        out_specs=[
            pl.BlockSpec((BP, D), lambda i: (i, 0)),
            pl.BlockSpec((K, BP, D), lambda i: (0, i, 0)),
        ],
        out_shape=[
            jax.ShapeDtypeStruct((N, D), jnp.float32),
            jax.ShapeDtypeStruct((K, N, D), jnp.float32),
        ],
    )(partials, h, wih_t, whh_t, bih2, bhh2, wt, b3d)


def _gru_mlp_body(p_ref, h_ref, wih_ref, whh_ref, bih_ref, bhh_ref,
                  w3_ref, b3_ref, w4_ref, b4_ref, out_ref):
    hn = _gru_core(p_ref, h_ref, wih_ref, whh_ref, bih_ref, bhh_ref, True)
    y = jnp.maximum(
        jnp.dot(hn, w3_ref[...], preferred_element_type=jnp.float32) + b3_ref[...],
        0.0)
    z = jnp.maximum(
        jnp.dot(y, w4_ref[...], preferred_element_type=jnp.float32) + b4_ref[...],
        0.0)
    out_ref[...] = z


def _tc_gru_mlp(partials, h, wih_t, whh_t, bih2, bhh2, w3t, b32, w4t, b42):
    return pl.pallas_call(
        _gru_mlp_body,
        grid=(N // BP,),
        in_specs=[
            pl.BlockSpec((NC, BP, D), lambda i: (0, i, 0)),
            pl.BlockSpec((BP, D), lambda i: (i, 0)),
            pl.BlockSpec((D, 3 * D), lambda i: (0, 0)),
            pl.BlockSpec((D, 3 * D), lambda i: (0, 0)),
            pl.BlockSpec((1, 3 * D), lambda i: (0, 0)),
            pl.BlockSpec((1, 3 * D), lambda i: (0, 0)),
            pl.BlockSpec((D, H), lambda i: (0, 0)),
            pl.BlockSpec((1, H), lambda i: (0, 0)),
            pl.BlockSpec((H, D), lambda i: (0, 0)),
            pl.BlockSpec((1, D), lambda i: (0, 0)),
        ],
        out_specs=pl.BlockSpec((BP, D), lambda i: (i, 0)),
        out_shape=jax.ShapeDtypeStruct((N, D), jnp.float32),
    )(partials, h, wih_t, whh_t, bih2, bhh2, w3t, b32, w4t, b42)


def _fin_body(p_ref, g_ref, out_ref):
    y = jnp.dot(p_ref[...], g_ref[...], preferred_element_type=jnp.float32)
    out_ref[...] = jax.nn.sigmoid(jax.nn.sigmoid(y))


def _tc_finish(partials, gmat):
    BF = 1728
    return pl.pallas_call(
        _fin_body,
        grid=(PROWS // BF,),
        in_specs=[
            pl.BlockSpec((BF, D), lambda i: (i, 0)),
            pl.BlockSpec((D, 8), lambda i: (0, 0)),
        ],
        out_specs=pl.BlockSpec((BF, 8), lambda i: (i, 0)),
        out_shape=jax.ShapeDtypeStruct((PROWS, 8), jnp.float32),
    )(partials, gmat)


def kernel(features, edge_index, edge_types, c1_W, c1_b, c1_Wih, c1_Whh,
           c1_bih, c1_bhh, c2_W, c2_b, c2_Wih, c2_Whh, c2_bih, c2_bhh,
           W3, b3, W4, b4):
    src = edge_index[0]
    dst = edge_index[1]
    gidx = edge_types * N + src

    npad = EPAD - E
    pad_ids = jnp.arange(npad, dtype=jnp.int32)
    gidx_p = jnp.concatenate([gidx, pad_ids % 997]).reshape(NW, NCH, CHUNK)
    dst_p = jnp.concatenate(
        [dst, N + (pad_ids % (NACC - N))]).reshape(NW, NCH, CHUNK)
    segidx = jnp.stack([gidx_p, dst_p], axis=2)
    src_sp = jnp.concatenate([src, pad_ids % 997]).reshape(NW, NCH, CHUNK)
    dst_sp = jnp.concatenate(
        [dst, (pad_ids + 499) % 997]).reshape(NW, NCH, CHUNK)
    scidx = jnp.stack([src_sp, dst_sp], axis=2)

    zeros_acc = jnp.zeros((NACC, D), jnp.float32)
    gmat = (jnp.arange(D, dtype=jnp.int32)[:, None] // L
            == jnp.arange(8, dtype=jnp.int32)[None, :]).astype(jnp.float32)

    wt1 = jnp.transpose(c1_W, (0, 2, 1))
    b1_3d = c1_b[:, None, :]
    wt2 = jnp.transpose(c2_W, (0, 2, 1))
    b2_3d = c2_b[:, None, :]
    g1 = (c1_Wih.T, c1_Whh.T, c1_bih[None, :], c1_bhh[None, :])
    g2 = (c2_Wih.T, c2_Whh.T, c2_bih[None, :], c2_bhh[None, :])

    h = features
    proj = _tc_proj(h, wt1, b1_3d)
    steps = [
        (g1, False, (wt1, b1_3d)),
        (g1, True, (wt2, b2_3d)),
        (g2, False, (wt2, b2_3d)),
        (g2, True, None),
    ]
    for gw, relu, nxt in steps:
        partials = _seg_sum(proj.reshape(K * N, D), segidx, zeros_acc)
        if nxt is None:
            x = _tc_gru_mlp(partials, h, *gw,
                            W3.T, b3[None, :], W4.T, b4[None, :])
        else:
            h, proj = _tc_gru_proj(partials, h, *gw, *nxt, relu)

    part = _edge_score(x, scidx)
    scores = _tc_finish(part, gmat).reshape(EPAD)
    return scores[:E]


# speedup vs baseline: 34.8379x; 1.0589x over previous
import functools

import jax
import jax.numpy as jnp
from jax import lax
from jax.experimental import pallas as pl
from jax.experimental.pallas import tpu as pltpu, tpu_sc as plsc

N, E, D, H, K = 10000, 320000, 128, 256, 4

NC, NS, L = 2, 16, 16
NW = NC * NS

CHUNK = 128
NCH = 81
EPW = NCH * CHUNK
EPAD = EPW * NW
NACC = 10112
RPT = NACC // NS
BP = 1000
PROWS = EPAD // 8

_SC_PARAMS = pltpu.CompilerParams(needs_layout_passes=False)


def _mesh():
    return plsc.VectorSubcoreMesh(
        core_axis_name="c", subcore_axis_name="s",
        num_cores=NC, num_subcores=NS)


def _seg_sum_body(proj_hbm, idx_hbm, zero_hbm, out_hbm,
                  ib0, ib1, ib2, r0, r1, r2, acc_sh,
                  sg0, sg1, sg2, si0, si1, si2, sc0, sc1, sc2):
    ibs = (ib0, ib1, ib2)
    rows = (r0, r1, r2)
    sg = (sg0, sg1, sg2)
    si = (si0, si1, si2)
    ssc = (sc0, sc1, sc2)
    c = lax.axis_index("c")
    s = lax.axis_index("s")
    wid = s * NC + c
    pltpu.sync_copy(zero_hbm.at[pl.ds(s * RPT, RPT)],
                    acc_sh.at[pl.ds(s * RPT, RPT)])
    plsc.subcore_barrier()

    pltpu.sync_copy(idx_hbm.at[wid, 0], ib0)
    pltpu.async_copy(proj_hbm.at[ib0.at[0]], r0, sg0)
    pltpu.async_copy(idx_hbm.at[wid, 1], ib1, si1)

    def body(i, carry):
        for b in range(3):
            cc = 3 * i + b
            nx = (b + 1) % 3
            pv = (b + 2) % 3

            @pl.when(cc + 1 < NCH)
            def _():
                pltpu.make_async_copy(idx_hbm.at[wid, cc + 1],
                                      ibs[nx], si[nx]).wait()
                pltpu.async_copy(proj_hbm.at[ibs[nx].at[0]], rows[nx], sg[nx])

            @pl.when(cc >= 1)
            def _():
                pltpu.make_async_copy(rows[pv], acc_sh.at[ibs[pv].at[1]],
                                      ssc[pv]).wait()

            @pl.when(cc + 2 < NCH)
            def _():
                pltpu.async_copy(idx_hbm.at[wid, cc + 2], ibs[pv], si[pv])

            pltpu.make_async_copy(proj_hbm.at[ibs[b].at[0]], rows[b],
                                  sg[b]).wait()
            pltpu.async_copy(rows[b], acc_sh.at[ibs[b].at[1]], ssc[b],
                             add=True)
        return carry

    lax.fori_loop(0, NCH // 3, body, 0)
    lsl = (NCH - 1) % 3
    pltpu.make_async_copy(rows[lsl], acc_sh.at[ibs[lsl].at[1]],
                          ssc[lsl]).wait()
    plsc.subcore_barrier()
    pltpu.sync_copy(acc_sh.at[pl.ds(s * RPT, RPT)],
                    out_hbm.at[c].at[pl.ds(s * RPT, RPT)])


@functools.cache
def _seg_sum_kernel():
    return pl.kernel(
        _seg_sum_body,
        out_type=jax.ShapeDtypeStruct((NC, NACC, D), jnp.float32),
        mesh=_mesh(),
        scratch_types=(
            [pltpu.VMEM((2, CHUNK), jnp.int32) for _ in range(3)]
            + [pltpu.VMEM((CHUNK, D), jnp.float32) for _ in range(3)]
            + [pltpu.VMEM_SHARED((NACC, D), jnp.float32)]
            + [pltpu.SemaphoreType.DMA for _ in range(9)]
        ),
        compiler_params=_SC_PARAMS,
    )


def _seg_sum(proj, segidx, zeros_acc):
    return _seg_sum_kernel()(proj, segidx, zeros_acc)


def _edge_score_body(x_hbm, idx_hbm, out_hbm,
                     si_v, rr0, rr1, rr2, p0, p1, p2,
                     sr0, sr1, sr2, so0, so1, so2):
    rrb = (rr0, rr1, rr2)
    pvs = (p0, p1, p2)
    srs = (sr0, sr1, sr2)
    sos = (so0, so1, so2)
    c = lax.axis_index("c")
    s = lax.axis_index("s")
    wid = s * NC + c
    pltpu.sync_copy(idx_hbm.at[wid], si_v)

    def _launch(cc, sl):
        pltpu.async_copy(x_hbm.at[si_v.at[cc].at[0]], rrb[sl].at[0], srs[sl])
        pltpu.async_copy(x_hbm.at[si_v.at[cc].at[1]], rrb[sl].at[1], srs[sl])

    def _wait(cc, sl):
        pltpu.make_async_copy(x_hbm.at[si_v.at[cc].at[0]], rrb[sl].at[0],
                              srs[sl]).wait()
        pltpu.make_async_copy(x_hbm.at[si_v.at[cc].at[1]], rrb[sl].at[1],
                              srs[sl]).wait()

    _launch(0, 0)
    _launch(1, 1)

    def body(i, carry):
        for b in range(3):
            cc = 3 * i + b
            pv = (b + 2) % 3

            @pl.when(cc + 2 < NCH)
            def _():
                _launch(cc + 2, pv)

            _wait(cc, b)

            @pl.when(cc >= 3)
            def _():
                pltpu.make_async_copy(
                    pvs[b], out_hbm.at[pl.ds(0, CHUNK // 8)], sos[b]).wait()

            rs, rd, p_v = rrb[b].at[0], rrb[b].at[1], pvs[b]

            def gbody(g, carry2):
                for j in range(8):
                    e = g * 8 + j
                    acc = rs[e, pl.ds(0, L)] * rd[e, pl.ds(0, L)]
                    for fc in range(1, 8):
                        acc = acc + (rs[e, pl.ds(fc * L, L)]
                                     * rd[e, pl.ds(fc * L, L)])
                    p_v[g, pl.ds(j * L, L)] = acc
                return carry2

            lax.fori_loop(0, CHUNK // 8, gbody, 0)
            base_row = wid * (EPW // 8) + cc * (CHUNK // 8)
            pltpu.async_copy(p_v, out_hbm.at[pl.ds(base_row, CHUNK // 8)],
                             sos[b])
        return carry

    lax.fori_loop(0, NCH // 3, body, 0)
    for b in range(3):
        pltpu.make_async_copy(pvs[b], out_hbm.at[pl.ds(0, CHUNK // 8)],
                              sos[b]).wait()


@functools.cache
def _edge_score_kernel():
    return pl.kernel(
        _edge_score_body,
        out_type=jax.ShapeDtypeStruct((PROWS, D), jnp.float32),
        mesh=_mesh(),
        scratch_types=(
            [pltpu.VMEM((NCH, 2, CHUNK), jnp.int32)]
            + [pltpu.VMEM((2, CHUNK, D), jnp.float32) for _ in range(3)]
            + [pltpu.VMEM((CHUNK // 8, D), jnp.float32) for _ in range(3)]
            + [pltpu.SemaphoreType.DMA for _ in range(6)]
        ),
        compiler_params=_SC_PARAMS,
    )


def _edge_score(x, scidx):
    return _edge_score_kernel()(x, scidx)


def _proj_body(h_ref, wt_ref, b_ref, out_ref):
    h = h_ref[...]
    for k in range(K):
        out_ref[k] = (jnp.dot(h, wt_ref[k], preferred_element_type=jnp.float32)
                      + b_ref[k])


def _tc_proj(h, wt, b3d):
    return pl.pallas_call(
        _proj_body,
        grid=(N // BP,),
        in_specs=[
            pl.BlockSpec((BP, D), lambda i: (i, 0)),
            pl.BlockSpec((K, D, D), lambda i: (0, 0, 0)),
            pl.BlockSpec((K, 1, D), lambda i: (0, 0, 0)),
        ],
        out_specs=pl.BlockSpec((K, BP, D), lambda i: (0, i, 0)),
        out_shape=jax.ShapeDtypeStruct((K, N, D), jnp.float32),
    )(h, wt, b3d)


def _gru_core(p_ref, h_ref, wih_ref, whh_ref, bih_ref, bhh_ref, relu):
    a = p_ref[0] + p_ref[1]
    h = h_ref[...]
    gi = jnp.dot(a, wih_ref[...], preferred_element_type=jnp.float32) + bih_ref[...]
    gh = jnp.dot(h, whh_ref[...], preferred_element_type=jnp.float32) + bhh_ref[...]
    r = jax.nn.sigmoid(gi[:, :D] + gh[:, :D])
    z = jax.nn.sigmoid(gi[:, D:2 * D] + gh[:, D:2 * D])
    n = jnp.tanh(gi[:, 2 * D:] + r * gh[:, 2 * D:])
    hn = (1.0 - z) * n + z * h
    if relu:
        hn = jnp.maximum(hn, 0.0)
    return hn


def _gru_proj_body(relu, p_ref, h_ref, wih_ref, whh_ref, bih_ref, bhh_ref,
                   wt_ref, b_ref, hn_ref, proj_ref):
    hn = _gru_core(p_ref, h_ref, wih_ref, whh_ref, bih_ref, bhh_ref, relu)
    hn_ref[...] = hn
    for k in range(K):
        proj_ref[k] = (jnp.dot(hn, wt_ref[k], preferred_element_type=jnp.float32)
                       + b_ref[k])


def _tc_gru_proj(partials, h, wih_t, whh_t, bih2, bhh2, wt, b3d, relu):
    return pl.pallas_call(
        functools.partial(_gru_proj_body, relu),
        grid=(N // BP,),
        in_specs=[
            pl.BlockSpec((NC, BP, D), lambda i: (0, i, 0)),
            pl.BlockSpec((BP, D), lambda i: (i, 0)),
            pl.BlockSpec((D, 3 * D), lambda i: (0, 0)),
            pl.BlockSpec((D, 3 * D), lambda i: (0, 0)),
            pl.BlockSpec((1, 3 * D), lambda i: (0, 0)),
            pl.BlockSpec((1, 3 * D), lambda i: (0, 0)),
            pl.BlockSpec((K, D, D), lambda i: (0, 0, 0)),
            pl.BlockSpec((K, 1, D), lambda i: (0, 0, 0)),
        ],
        out_specs=[
            pl.BlockSpec((BP, D), lambda i: (i, 0)),
            pl.BlockSpec((K, BP, D), lambda i: (0, i, 0)),
        ],
        out_shape=[
            jax.ShapeDtypeStruct((N, D), jnp.float32),
            jax.ShapeDtypeStruct((K, N, D), jnp.float32),
        ],
    )(partials, h, wih_t, whh_t, bih2, bhh2, wt, b3d)


def _gru_mlp_body(p_ref, h_ref, wih_ref, whh_ref, bih_ref, bhh_ref,
                  w3_ref, b3_ref, w4_ref, b4_ref, out_ref):
    hn = _gru_core(p_ref, h_ref, wih_ref, whh_ref, bih_ref, bhh_ref, True)
    y = jnp.maximum(
        jnp.dot(hn, w3_ref[...], preferred_element_type=jnp.float32) + b3_ref[...],
        0.0)
    z = jnp.maximum(
        jnp.dot(y, w4_ref[...], preferred_element_type=jnp.float32) + b4_ref[...],
        0.0)
    out_ref[...] = z


def _tc_gru_mlp(partials, h, wih_t, whh_t, bih2, bhh2, w3t, b32, w4t, b42):
    return pl.pallas_call(
        _gru_mlp_body,
        grid=(N // BP,),
        in_specs=[
            pl.BlockSpec((NC, BP, D), lambda i: (0, i, 0)),
            pl.BlockSpec((BP, D), lambda i: (i, 0)),
            pl.BlockSpec((D, 3 * D), lambda i: (0, 0)),
            pl.BlockSpec((D, 3 * D), lambda i: (0, 0)),
            pl.BlockSpec((1, 3 * D), lambda i: (0, 0)),
            pl.BlockSpec((1, 3 * D), lambda i: (0, 0)),
            pl.BlockSpec((D, H), lambda i: (0, 0)),
            pl.BlockSpec((1, H), lambda i: (0, 0)),
            pl.BlockSpec((H, D), lambda i: (0, 0)),
            pl.BlockSpec((1, D), lambda i: (0, 0)),
        ],
        out_specs=pl.BlockSpec((BP, D), lambda i: (i, 0)),
        out_shape=jax.ShapeDtypeStruct((N, D), jnp.float32),
    )(partials, h, wih_t, whh_t, bih2, bhh2, w3t, b32, w4t, b42)


def _fin_body(p_ref, g_ref, out_ref):
    y = jnp.dot(p_ref[...], g_ref[...], preferred_element_type=jnp.float32)
    out_ref[...] = jax.nn.sigmoid(jax.nn.sigmoid(y))


def _tc_finish(partials, gmat):
    BF = 1728
    return pl.pallas_call(
        _fin_body,
        grid=(PROWS // BF,),
        in_specs=[
            pl.BlockSpec((BF, D), lambda i: (i, 0)),
            pl.BlockSpec((D, 8), lambda i: (0, 0)),
        ],
        out_specs=pl.BlockSpec((BF, 8), lambda i: (i, 0)),
        out_shape=jax.ShapeDtypeStruct((PROWS, 8), jnp.float32),
    )(partials, gmat)


def kernel(features, edge_index, edge_types, c1_W, c1_b, c1_Wih, c1_Whh,
           c1_bih, c1_bhh, c2_W, c2_b, c2_Wih, c2_Whh, c2_bih, c2_bhh,
           W3, b3, W4, b4):
    src = edge_index[0]
    dst = edge_index[1]
    gidx = edge_types * N + src

    npad = EPAD - E
    pad_ids = jnp.arange(npad, dtype=jnp.int32)
    gidx_p = jnp.concatenate([gidx, pad_ids % 997]).reshape(NW, NCH, CHUNK)
    dst_p = jnp.concatenate(
        [dst, N + (pad_ids % (NACC - N))]).reshape(NW, NCH, CHUNK)
    segidx = jnp.stack([gidx_p, dst_p], axis=2)
    src_sp = jnp.concatenate([src, pad_ids % 997]).reshape(NW, NCH, CHUNK)
    dst_sp = jnp.concatenate(
        [dst, (pad_ids + 499) % 997]).reshape(NW, NCH, CHUNK)
    scidx = jnp.stack([src_sp, dst_sp], axis=2)

    zeros_acc = jnp.zeros((NACC, D), jnp.float32)
    gmat = (jnp.arange(D, dtype=jnp.int32)[:, None] // L
            == jnp.arange(8, dtype=jnp.int32)[None, :]).astype(jnp.float32)

    wt1 = jnp.transpose(c1_W, (0, 2, 1))
    b1_3d = c1_b[:, None, :]
    wt2 = jnp.transpose(c2_W, (0, 2, 1))
    b2_3d = c2_b[:, None, :]
    g1 = (c1_Wih.T, c1_Whh.T, c1_bih[None, :], c1_bhh[None, :])
    g2 = (c2_Wih.T, c2_Whh.T, c2_bih[None, :], c2_bhh[None, :])

    h = features
    proj = _tc_proj(h, wt1, b1_3d)
    steps = [
        (g1, False, (wt1, b1_3d)),
        (g1, True, (wt2, b2_3d)),
        (g2, False, (wt2, b2_3d)),
        (g2, True, None),
    ]
    for gw, relu, nxt in steps:
        partials = _seg_sum(proj.reshape(K * N, D), segidx, zeros_acc)
        if nxt is None:
            x = _tc_gru_mlp(partials, h, *gw,
                            W3.T, b3[None, :], W4.T, b4[None, :])
        else:
            h, proj = _tc_gru_proj(partials, h, *gw, *nxt, relu)

    part = _edge_score(x, scidx)
    scores = _tc_finish(part, gmat).reshape(EPAD)
    return scores[:E]


# speedup vs baseline: 35.4186x; 1.0167x over previous
import functools

import jax
import jax.numpy as jnp
from jax import lax
from jax.experimental import pallas as pl
from jax.experimental.pallas import tpu as pltpu, tpu_sc as plsc

N, E, D, H, K = 10000, 320000, 128, 256, 4

NC, NS, L = 2, 16, 16
NW = NC * NS

CHUNK = 128
NCH = 81
EPW = NCH * CHUNK
EPAD = EPW * NW
NACC = 10112
RPT = NACC // NS
BP = 2000
PROWS = EPAD // 8

_SC_PARAMS = pltpu.CompilerParams(needs_layout_passes=False)


def _mesh():
    return plsc.VectorSubcoreMesh(
        core_axis_name="c", subcore_axis_name="s",
        num_cores=NC, num_subcores=NS)


def _seg_sum_body(proj_hbm, idx_hbm, zero_hbm, out_hbm,
                  ib0, ib1, ib2, r0, r1, r2, acc_sh,
                  sg0, sg1, sg2, si0, si1, si2, sc0, sc1, sc2):
    ibs = (ib0, ib1, ib2)
    rows = (r0, r1, r2)
    sg = (sg0, sg1, sg2)
    si = (si0, si1, si2)
    ssc = (sc0, sc1, sc2)
    c = lax.axis_index("c")
    s = lax.axis_index("s")
    wid = s * NC + c
    pltpu.sync_copy(zero_hbm.at[pl.ds(s * RPT, RPT)],
                    acc_sh.at[pl.ds(s * RPT, RPT)])
    plsc.subcore_barrier()

    pltpu.sync_copy(idx_hbm.at[wid, 0], ib0)
    pltpu.async_copy(proj_hbm.at[ib0.at[0]], r0, sg0)
    pltpu.async_copy(idx_hbm.at[wid, 1], ib1, si1)

    def body(i, carry):
        for b in range(3):
            cc = 3 * i + b
            nx = (b + 1) % 3
            pv = (b + 2) % 3

            @pl.when(cc + 1 < NCH)
            def _():
                pltpu.make_async_copy(idx_hbm.at[wid, cc + 1],
                                      ibs[nx], si[nx]).wait()
                pltpu.async_copy(proj_hbm.at[ibs[nx].at[0]], rows[nx], sg[nx])

            @pl.when(cc >= 1)
            def _():
                pltpu.make_async_copy(rows[pv], acc_sh.at[ibs[pv].at[1]],
                                      ssc[pv]).wait()

            @pl.when(cc + 2 < NCH)
            def _():
                pltpu.async_copy(idx_hbm.at[wid, cc + 2], ibs[pv], si[pv])

            pltpu.make_async_copy(proj_hbm.at[ibs[b].at[0]], rows[b],
                                  sg[b]).wait()
            pltpu.async_copy(rows[b], acc_sh.at[ibs[b].at[1]], ssc[b],
                             add=True)
        return carry

    lax.fori_loop(0, NCH // 3, body, 0)
    lsl = (NCH - 1) % 3
    pltpu.make_async_copy(rows[lsl], acc_sh.at[ibs[lsl].at[1]],
                          ssc[lsl]).wait()
    plsc.subcore_barrier()
    pltpu.sync_copy(acc_sh.at[pl.ds(s * RPT, RPT)],
                    out_hbm.at[c].at[pl.ds(s * RPT, RPT)])


@functools.cache
def _seg_sum_kernel():
    return pl.kernel(
        _seg_sum_body,
        out_type=jax.ShapeDtypeStruct((NC, NACC, D), jnp.float32),
        mesh=_mesh(),
        scratch_types=(
            [pltpu.VMEM((2, CHUNK), jnp.int32) for _ in range(3)]
            + [pltpu.VMEM((CHUNK, D), jnp.float32) for _ in range(3)]
            + [pltpu.VMEM_SHARED((NACC, D), jnp.float32)]
            + [pltpu.SemaphoreType.DMA for _ in range(9)]
        ),
        compiler_params=_SC_PARAMS,
    )


def _seg_sum(proj, segidx, zeros_acc):
    return _seg_sum_kernel()(proj, segidx, zeros_acc)


def _edge_score_body(x_hbm, idx_hbm, out_hbm,
                     si_v, rr0, rr1, rr2, p0, p1, p2,
                     sr0, sr1, sr2, so0, so1, so2):
    rrb = (rr0, rr1, rr2)
    pvs = (p0, p1, p2)
    srs = (sr0, sr1, sr2)
    sos = (so0, so1, so2)
    c = lax.axis_index("c")
    s = lax.axis_index("s")
    wid = s * NC + c
    pltpu.sync_copy(idx_hbm.at[wid], si_v)

    def _launch(cc, sl):
        pltpu.async_copy(x_hbm.at[si_v.at[cc].at[0]], rrb[sl].at[0], srs[sl])
        pltpu.async_copy(x_hbm.at[si_v.at[cc].at[1]], rrb[sl].at[1], srs[sl])

    def _wait(cc, sl):
        pltpu.make_async_copy(x_hbm.at[si_v.at[cc].at[0]], rrb[sl].at[0],
                              srs[sl]).wait()
        pltpu.make_async_copy(x_hbm.at[si_v.at[cc].at[1]], rrb[sl].at[1],
                              srs[sl]).wait()

    _launch(0, 0)
    _launch(1, 1)

    def body(i, carry):
        for b in range(3):
            cc = 3 * i + b
            pv = (b + 2) % 3

            @pl.when(cc + 2 < NCH)
            def _():
                _launch(cc + 2, pv)

            _wait(cc, b)

            @pl.when(cc >= 3)
            def _():
                pltpu.make_async_copy(
                    pvs[b], out_hbm.at[pl.ds(0, CHUNK // 8)], sos[b]).wait()

            rs, rd, p_v = rrb[b].at[0], rrb[b].at[1], pvs[b]

            def gbody(g, carry2):
                for j in range(8):
                    e = g * 8 + j
                    acc = rs[e, pl.ds(0, L)] * rd[e, pl.ds(0, L)]
                    for fc in range(1, 8):
                        acc = acc + (rs[e, pl.ds(fc * L, L)]
                                     * rd[e, pl.ds(fc * L, L)])
                    p_v[g, pl.ds(j * L, L)] = acc
                return carry2

            lax.fori_loop(0, CHUNK // 8, gbody, 0)
            base_row = wid * (EPW // 8) + cc * (CHUNK // 8)
            pltpu.async_copy(p_v, out_hbm.at[pl.ds(base_row, CHUNK // 8)],
                             sos[b])
        return carry

    lax.fori_loop(0, NCH // 3, body, 0)
    for b in range(3):
        pltpu.make_async_copy(pvs[b], out_hbm.at[pl.ds(0, CHUNK // 8)],
                              sos[b]).wait()


@functools.cache
def _edge_score_kernel():
    return pl.kernel(
        _edge_score_body,
        out_type=jax.ShapeDtypeStruct((PROWS, D), jnp.float32),
        mesh=_mesh(),
        scratch_types=(
            [pltpu.VMEM((NCH, 2, CHUNK), jnp.int32)]
            + [pltpu.VMEM((2, CHUNK, D), jnp.float32) for _ in range(3)]
            + [pltpu.VMEM((CHUNK // 8, D), jnp.float32) for _ in range(3)]
            + [pltpu.SemaphoreType.DMA for _ in range(6)]
        ),
        compiler_params=_SC_PARAMS,
    )


def _edge_score(x, scidx):
    return _edge_score_kernel()(x, scidx)


def _proj_body(h_ref, wt_ref, b_ref, out_ref):
    h = h_ref[...]
    for k in range(K):
        out_ref[k] = (jnp.dot(h, wt_ref[k], preferred_element_type=jnp.float32)
                      + b_ref[k])


def _tc_proj(h, wt, b3d):
    return pl.pallas_call(
        _proj_body,
        grid=(N // BP,),
        in_specs=[
            pl.BlockSpec((BP, D), lambda i: (i, 0)),
            pl.BlockSpec((K, D, D), lambda i: (0, 0, 0)),
            pl.BlockSpec((K, 1, D), lambda i: (0, 0, 0)),
        ],
        out_specs=pl.BlockSpec((K, BP, D), lambda i: (0, i, 0)),
        out_shape=jax.ShapeDtypeStruct((K, N, D), jnp.float32),
    )(h, wt, b3d)


def _gru_core(p_ref, h_ref, wih_ref, whh_ref, bih_ref, bhh_ref, relu):
    a = p_ref[0] + p_ref[1]
    h = h_ref[...]
    gi = jnp.dot(a, wih_ref[...], preferred_element_type=jnp.float32) + bih_ref[...]
    gh = jnp.dot(h, whh_ref[...], preferred_element_type=jnp.float32) + bhh_ref[...]
    r = jax.nn.sigmoid(gi[:, :D] + gh[:, :D])
    z = jax.nn.sigmoid(gi[:, D:2 * D] + gh[:, D:2 * D])
    n = jnp.tanh(gi[:, 2 * D:] + r * gh[:, 2 * D:])
    hn = (1.0 - z) * n + z * h
    if relu:
        hn = jnp.maximum(hn, 0.0)
    return hn


def _gru_proj_body(relu, p_ref, h_ref, wih_ref, whh_ref, bih_ref, bhh_ref,
                   wt_ref, b_ref, hn_ref, proj_ref):
    hn = _gru_core(p_ref, h_ref, wih_ref, whh_ref, bih_ref, bhh_ref, relu)
    hn_ref[...] = hn
    for k in range(K):
        proj_ref[k] = (jnp.dot(hn, wt_ref[k], preferred_element_type=jnp.float32)
                       + b_ref[k])


def _tc_gru_proj(partials, h, wih_t, whh_t, bih2, bhh2, wt, b3d, relu):
    return pl.pallas_call(
        functools.partial(_gru_proj_body, relu),
        grid=(N // BP,),
        in_specs=[
            pl.BlockSpec((NC, BP, D), lambda i: (0, i, 0)),
            pl.BlockSpec((BP, D), lambda i: (i, 0)),
            pl.BlockSpec((D, 3 * D), lambda i: (0, 0)),
            pl.BlockSpec((D, 3 * D), lambda i: (0, 0)),
            pl.BlockSpec((1, 3 * D), lambda i: (0, 0)),
            pl.BlockSpec((1, 3 * D), lambda i: (0, 0)),
            pl.BlockSpec((K, D, D), lambda i: (0, 0, 0)),
            pl.BlockSpec((K, 1, D), lambda i: (0, 0, 0)),
        ],
        out_specs=[
            pl.BlockSpec((BP, D), lambda i: (i, 0)),
            pl.BlockSpec((K, BP, D), lambda i: (0, i, 0)),
        ],
        out_shape=[
            jax.ShapeDtypeStruct((N, D), jnp.float32),
            jax.ShapeDtypeStruct((K, N, D), jnp.float32),
        ],
    )(partials, h, wih_t, whh_t, bih2, bhh2, wt, b3d)


def _gru_mlp_body(p_ref, h_ref, wih_ref, whh_ref, bih_ref, bhh_ref,
                  w3_ref, b3_ref, w4_ref, b4_ref, out_ref):
    hn = _gru_core(p_ref, h_ref, wih_ref, whh_ref, bih_ref, bhh_ref, True)
    y = jnp.maximum(
        jnp.dot(hn, w3_ref[...], preferred_element_type=jnp.float32) + b3_ref[...],
        0.0)
    z = jnp.maximum(
        jnp.dot(y, w4_ref[...], preferred_element_type=jnp.float32) + b4_ref[...],
        0.0)
    out_ref[...] = z


def _tc_gru_mlp(partials, h, wih_t, whh_t, bih2, bhh2, w3t, b32, w4t, b42):
    return pl.pallas_call(
        _gru_mlp_body,
        grid=(N // BP,),
        in_specs=[
            pl.BlockSpec((NC, BP, D), lambda i: (0, i, 0)),
            pl.BlockSpec((BP, D), lambda i: (i, 0)),
            pl.BlockSpec((D, 3 * D), lambda i: (0, 0)),
            pl.BlockSpec((D, 3 * D), lambda i: (0, 0)),
            pl.BlockSpec((1, 3 * D), lambda i: (0, 0)),
            pl.BlockSpec((1, 3 * D), lambda i: (0, 0)),
            pl.BlockSpec((D, H), lambda i: (0, 0)),
            pl.BlockSpec((1, H), lambda i: (0, 0)),
            pl.BlockSpec((H, D), lambda i: (0, 0)),
            pl.BlockSpec((1, D), lambda i: (0, 0)),
        ],
        out_specs=pl.BlockSpec((BP, D), lambda i: (i, 0)),
        out_shape=jax.ShapeDtypeStruct((N, D), jnp.float32),
    )(partials, h, wih_t, whh_t, bih2, bhh2, w3t, b32, w4t, b42)


def _fin_body(p_ref, g_ref, out_ref):
    y = jnp.dot(p_ref[...], g_ref[...], preferred_element_type=jnp.float32)
    out_ref[...] = jax.nn.sigmoid(jax.nn.sigmoid(y))


def _tc_finish(partials, gmat):
    BF = 1728
    return pl.pallas_call(
        _fin_body,
        grid=(PROWS // BF,),
        in_specs=[
            pl.BlockSpec((BF, D), lambda i: (i, 0)),
            pl.BlockSpec((D, 8), lambda i: (0, 0)),
        ],
        out_specs=pl.BlockSpec((BF, 8), lambda i: (i, 0)),
        out_shape=jax.ShapeDtypeStruct((PROWS, 8), jnp.float32),
    )(partials, gmat)


def kernel(features, edge_index, edge_types, c1_W, c1_b, c1_Wih, c1_Whh,
           c1_bih, c1_bhh, c2_W, c2_b, c2_Wih, c2_Whh, c2_bih, c2_bhh,
           W3, b3, W4, b4):
    src = edge_index[0]
    dst = edge_index[1]
    gidx = edge_types * N + src

    npad = EPAD - E
    pad_ids = jnp.arange(npad, dtype=jnp.int32)
    gidx_p = jnp.concatenate([gidx, pad_ids % 997]).reshape(NW, NCH, CHUNK)
    dst_p = jnp.concatenate(
        [dst, N + (pad_ids % (NACC - N))]).reshape(NW, NCH, CHUNK)
    segidx = jnp.stack([gidx_p, dst_p], axis=2)
    src_sp = jnp.concatenate([src, pad_ids % 997]).reshape(NW, NCH, CHUNK)
    dst_sp = jnp.concatenate(
        [dst, (pad_ids + 499) % 997]).reshape(NW, NCH, CHUNK)
    scidx = jnp.stack([src_sp, dst_sp], axis=2)

    zeros_acc = jnp.zeros((NACC, D), jnp.float32)
    gmat = (jnp.arange(D, dtype=jnp.int32)[:, None] // L
            == jnp.arange(8, dtype=jnp.int32)[None, :]).astype(jnp.float32)

    wt1 = jnp.transpose(c1_W, (0, 2, 1))
    b1_3d = c1_b[:, None, :]
    wt2 = jnp.transpose(c2_W, (0, 2, 1))
    b2_3d = c2_b[:, None, :]
    g1 = (c1_Wih.T, c1_Whh.T, c1_bih[None, :], c1_bhh[None, :])
    g2 = (c2_Wih.T, c2_Whh.T, c2_bih[None, :], c2_bhh[None, :])

    h = features
    proj = _tc_proj(h, wt1, b1_3d)
    steps = [
        (g1, False, (wt1, b1_3d)),
        (g1, True, (wt2, b2_3d)),
        (g2, False, (wt2, b2_3d)),
        (g2, True, None),
    ]
    for gw, relu, nxt in steps:
        partials = _seg_sum(proj.reshape(K * N, D), segidx, zeros_acc)
        if nxt is None:
            x = _tc_gru_mlp(partials, h, *gw,
                            W3.T, b3[None, :], W4.T, b4[None, :])
        else:
            h, proj = _tc_gru_proj(partials, h, *gw, *nxt, relu)

    part = _edge_score(x, scidx)
    scores = _tc_finish(part, gmat).reshape(EPAD)
    return scores[:E]
